# Initial kernel scaffold; baseline (speedup 1.0000x reference)
import jax
import jax.numpy as jnp
from jax.experimental import pallas as pl


def kernel(T, time_feature, edge_index, W_openHist, b_openHist, w_hist, b_hist, w_time, b_time, W_state, b_state, KR1_W, KR1_b, KR2_W, KR2_b, KRU0_W, KRU0_b, Kappa, HE_W, HE_b, RS_W, RS_b, C0, C1, W_close, b_close):
    raise NotImplementedError("write your pallas kernel here")


# speedup vs baseline: 8.5360x; 8.5360x over previous
import functools

import jax
import jax.numpy as jnp
from jax import lax
from jax.experimental import pallas as pl
from jax.experimental.pallas import tpu as pltpu
from jax.experimental.pallas import tpu_sc as plsc

NC = 2
NS = 16
L = 16

H = 0.1
S0 = float((1.0 + 1e-5) ** -0.5)

STEP = 128


def _round_up(x, m):
    return (x + m - 1) // m * m


def _fill(buf, val):
    v = jnp.full((L,), val, jnp.float32)

    def body(i, _):
        buf[i, pl.ds(0, L)] = v
        buf[i, pl.ds(L, L)] = v
        return 0
    lax.fori_loop(0, buf.shape[0], body, 0)


def _deg_kernel(stripe, nsteps, src_i, deg_out, acc, idxb, onesb):
    c = lax.axis_index("c")
    s = lax.axis_index("s")
    _fill(onesb, 0.0)
    for z in range(stripe // 128):
        pltpu.sync_copy(onesb, acc.at[pl.ds(s * stripe + z * 128, 128)])
    _fill(onesb, 1.0)
    plsc.subcore_barrier()

    ebase = (c * NS + s) * nsteps * STEP

    def step(j, _):
        pltpu.sync_copy(src_i.at[pl.ds(ebase + j * STEP, STEP)], idxb)
        pltpu.sync_copy(onesb, acc.at[idxb], add=True)
        return 0
    lax.fori_loop(0, nsteps, step, 0)

    plsc.subcore_barrier()
    r0 = s * stripe

    def cp(z, _):
        pltpu.sync_copy(acc.at[pl.ds(r0 + z * 128, 128)], onesb)
        pltpu.sync_copy(onesb, deg_out.at[c, pl.ds(r0 + z * 128, 128)])
        return 0
    lax.fori_loop(0, stripe // 128, cp, 0)


@functools.partial(jax.jit, static_argnums=(1,))
def _deg_call(src_i, nrows):
    stripe = nrows // NS
    nsteps = src_i.shape[0] // (NC * NS * STEP)
    mesh = plsc.VectorSubcoreMesh(core_axis_name="c", subcore_axis_name="s",
                                  num_cores=NC, num_subcores=NS)
    kern = pl.kernel(
        functools.partial(_deg_kernel, stripe, nsteps),
        out_type=jax.ShapeDtypeStruct((NC, nrows, 32), jnp.float32),
        mesh=mesh,
        scratch_types=[
            pltpu.VMEM_SHARED((nrows, 32), jnp.float32),
            pltpu.VMEM((STEP,), jnp.int32),
            pltpu.VMEM((128, 32), jnp.float32),
        ],
        compiler_params=pltpu.CompilerParams(use_tc_tiling_on_sc=False),
    )
    return kern(src_i)


def _lap_kernel(stripe, nsteps, table, src_i, dst_i, out, acc, srcb, dstb,
                rows0, rows1, sem0):
    c = lax.axis_index("c")
    s = lax.axis_index("s")

    _fill(rows0, 0.0)
    for z in range(stripe // 128):
        pltpu.sync_copy(rows0, acc.at[pl.ds(s * stripe + z * 128, 128)])
    plsc.subcore_barrier()

    ebase = s * nsteps * STEP

    def step(j, _):
        off = ebase + j * STEP
        pltpu.sync_copy(src_i.at[pl.ds(off, STEP)], srcb)
        pltpu.sync_copy(dst_i.at[pl.ds(off, STEP)], dstb)
        pltpu.async_copy(table.at[c].at[srcb], rows0, sem0).wait()
        pltpu.sync_copy(rows0, acc.at[dstb], add=True)
        return 0
    lax.fori_loop(0, nsteps, step, 0)

    plsc.subcore_barrier()
    r0 = s * stripe

    def cp(z, _):
        pltpu.sync_copy(acc.at[pl.ds(r0 + z * 128, 128)], rows1)
        pltpu.sync_copy(rows1, out.at[c, pl.ds(r0 + z * 128, 128)])
        return 0
    lax.fori_loop(0, stripe // 128, cp, 0)


@functools.partial(jax.jit, static_argnums=(3,))
def _lap_call(table, src_i, dst_i, acc_rows):
    stripe = acc_rows // NS
    hw = table.shape[2]
    nsteps = src_i.shape[0] // (NS * STEP)
    mesh = plsc.VectorSubcoreMesh(core_axis_name="c", subcore_axis_name="s",
                                  num_cores=NC, num_subcores=NS)
    kern = pl.kernel(
        functools.partial(_lap_kernel, stripe, nsteps),
        out_type=jax.ShapeDtypeStruct((NC, acc_rows, hw), jnp.float32),
        mesh=mesh,
        scratch_types=[
            pltpu.VMEM_SHARED((acc_rows, hw), jnp.float32),
            pltpu.VMEM((STEP,), jnp.int32),
            pltpu.VMEM((STEP,), jnp.int32),
            pltpu.VMEM((128, hw), jnp.float32),
            pltpu.VMEM((128, hw), jnp.float32),
            pltpu.SemaphoreType.DMA,
        ],
        compiler_params=pltpu.CompilerParams(use_tc_tiling_on_sc=False),
    )
    return kern(table, src_i, dst_i)


def _relu_bn(x):
    return jnp.maximum(x * S0, 0.0)


def _dinv_of(deg_r):
    deg = deg_r[0, 0] + deg_r[1, 0]
    return jnp.where(deg > 0, lax.rsqrt(jnp.maximum(deg, 1.0)), 0.0)


def _prologue(Tb, tfb, WoT, bo, wh, bh, wt, bt, WsT, bs):
    Thist = _relu_bn(jnp.dot(Tb, WoT, preferred_element_type=jnp.float32) + bo)
    Tst = _relu_bn(Tb[:, -1:] * wh + bh)
    pre = jnp.dot(tfb, wt, preferred_element_type=jnp.float32) + bt
    te = pre * jax.nn.sigmoid(pre)
    Tst = Tst + te * WsT + bs
    return Thist, Tst


def _reaction(Thist, comb, HEaT, HEbT, heb, K1T, k1b, K2T, k2b, KUT, kub):
    Th = _relu_bn(jnp.dot(Thist, HEaT, preferred_element_type=jnp.float32)
                  + jnp.dot(comb, HEbT, preferred_element_type=jnp.float32) + heb)
    dT = (jnp.dot(Th, K1T, preferred_element_type=jnp.float32) + k1b
          + jnp.dot(Thist, KUT, preferred_element_type=jnp.float32) + kub
          + comb * jnp.clip(jnp.dot(Th, K2T, preferred_element_type=jnp.float32)
                            + k2b, -1.0, 1.0))
    return _relu_bn(comb + H * dT)


def _diffusion(Rst, ay_r, dinv, Kd):
    ay = jnp.concatenate([ay_r[0, 0], ay_r[1, 0]], axis=-1) * dinv
    return Rst - H * Kd * (Rst - ay)


def _tc1_body(T_r, tf_r, deg_r, WoT_r, bo_r, wh_r, bh_r, wt_r, bt_r, WsT_r,
              bs_r, C0_r, HEaT_r, HEbT_r, heb_r, K1T_r, k1b_r, K2T_r, k2b_r,
              KUT_r, kub_r, rst_out, rs2_out):
    Tb = T_r[...]
    Thist, Tst = _prologue(Tb, tf_r[...], WoT_r[...], bo_r[...], wh_r[...],
                           bh_r[...], wt_r[...], bt_r[...], WsT_r[...],
                           bs_r[...])
    comb = Tst * C0_r[:, 0:1]
    Rst = _reaction(Thist, comb, HEaT_r[...], HEbT_r[...], heb_r[...],
                    K1T_r[...], k1b_r[...], K2T_r[...], k2b_r[...],
                    KUT_r[...], kub_r[...])
    rst_out[...] = Rst
    dinv = _dinv_of(deg_r)
    Rs = Rst * dinv
    rs2_out[0] = Rs[:, :32]
    rs2_out[1] = Rs[:, 32:]


def _tc2_body(T_r, tf_r, deg_r, rst0_r, ay_r, WoT_r, bo_r, wh_r, bh_r, wt_r,
              bt_r, WsT_r, bs_r, C1_r, Kd0_r, RSaT_r, RSbT_r, rsb_r, HEaT_r,
              HEbT_r, heb_r, K1T_r, k1b_r, K2T_r, k2b_r, KUT_r, kub_r,
              rst_out, rs2_out):
    Tb = T_r[...]
    dinv = _dinv_of(deg_r)
    Dst = _diffusion(rst0_r[...], ay_r, dinv, Kd0_r[...])
    Tnew1 = _relu_bn(jnp.dot(Tb, RSaT_r[...], preferred_element_type=jnp.float32)
                     + jnp.dot(Dst, RSbT_r[...], preferred_element_type=jnp.float32)
                     + rsb_r[...])
    Thist, Tst = _prologue(Tb, tf_r[...], WoT_r[...], bo_r[...], wh_r[...],
                           bh_r[...], wt_r[...], bt_r[...], WsT_r[...],
                           bs_r[...])
    comb = Tst * C1_r[:, 0:1] + Tnew1 * C1_r[:, 1:2]
    Rst = _reaction(Thist, comb, HEaT_r[...], HEbT_r[...], heb_r[...],
                    K1T_r[...], k1b_r[...], K2T_r[...], k2b_r[...],
                    KUT_r[...], kub_r[...])
    rst_out[...] = Rst
    Rs = Rst * dinv
    rs2_out[0] = Rs[:, :32]
    rs2_out[1] = Rs[:, 32:]


def _tc3_body(T_r, deg_r, rst1_r, ay_r, Kd1_r, RSaT_r, RSbT_r, rsb_r, WcT_r,
              bc_r, out_r):
    Tb = T_r[...]
    dinv = _dinv_of(deg_r)
    Dst = _diffusion(rst1_r[...], ay_r, dinv, Kd1_r[...])
    Tnew2 = _relu_bn(jnp.dot(Tb, RSaT_r[...], preferred_element_type=jnp.float32)
                     + jnp.dot(Dst, RSbT_r[...], preferred_element_type=jnp.float32)
                     + rsb_r[...])
    out_r[...] = (jnp.dot(Tnew2, WcT_r[...], preferred_element_type=jnp.float32)
                  + bc_r[...])


def _wspec(shape):
    nd = len(shape)
    return pl.BlockSpec(shape, lambda i: (0,) * nd)


def _rowspec(nb, d):
    return pl.BlockSpec((nb, d), lambda i: (i, 0))


def _degspec(nb):
    return pl.BlockSpec((2, 1, nb, 1), lambda i: (0, i, 0, 0))


def _ayspec(nb, hw):
    return pl.BlockSpec((2, 1, nb, hw), lambda i: (0, i, 0, 0))


def kernel(T, time_feature, edge_index, W_openHist, b_openHist, w_hist, b_hist,
           w_time, b_time, W_state, b_state, KR1_W, KR1_b, KR2_W, KR2_b,
           KRU0_W, KRU0_b, Kappa, HE_W, HE_b, RS_W, RS_b, C0, C1, W_close,
           b_close):
    N = T.shape[0]
    E = edge_index.shape[1]
    nin = T.shape[1]
    nhid = W_openHist.shape[0]
    hw = nhid // 2

    NB = 1000
    while N % NB:
        NB -= 8
    nblk = N // NB

    e_pad = _round_up(E, NC * NS * STEP)
    acc_rows = _round_up(N + 1, NS * 128)
    stripe = acc_rows // NS

    src = edge_index[0]
    dst = edge_index[1]
    pad = e_pad - E
    src_g = jnp.concatenate([src, jnp.zeros((pad,), jnp.int32)])
    dst_g = jnp.concatenate([dst, jnp.full((pad,), N, jnp.int32)])
    src_d = jnp.concatenate([src, jnp.full((pad,), N, jnp.int32)])

    f32 = jnp.float32
    WoT = W_openHist.T
    bo = b_openHist[None]
    wh = w_hist[None]
    bh = b_hist[None]
    wt = w_time[:, None]
    bt = b_time[None]
    WsT = W_state.T
    bs = b_state[None]
    C0m = C0[None].astype(f32)
    C1m = C1[None].astype(f32)
    Kd = jnp.clip(Kappa, 0.0, 1.0)
    HEaT = [(HE_W[i][:, :nhid] + HE_W[i][:, 2 * nhid:]).T for i in range(2)]
    HEbT = [HE_W[i][:, nhid:2 * nhid].T for i in range(2)]
    heb = [HE_b[i][None] for i in range(2)]
    K1T = [KR1_W[i].T for i in range(2)]
    k1b = [KR1_b[i][None] for i in range(2)]
    K2T = [KR2_W[i].T for i in range(2)]
    k2b = [KR2_b[i][None] for i in range(2)]
    KUT = [KRU0_W[i].T for i in range(2)]
    kub = [KRU0_b[i][None] for i in range(2)]
    RSaT = [RS_W[i][:, :nin].T for i in range(2)]
    RSbT = [RS_W[i][:, nin:].T for i in range(2)]
    rsb = [RS_b[i][None] for i in range(2)]
    WcT = W_close.T
    bc = b_close[None]

    tf2 = time_feature.reshape(N, -1)
    nfreq = tf2.shape[1]

    deg_p = _deg_call(src_d, acc_rows)
    deg_tc = deg_p[:, :N, 0:1].reshape(2, nblk, NB, 1)

    cparams = pltpu.CompilerParams(dimension_semantics=("arbitrary",))

    w1 = [WoT, bo, wh, bh, wt, bt, WsT, bs, C0m, HEaT[0], HEbT[0], heb[0],
          K1T[0], k1b[0], K2T[0], k2b[0], KUT[0], kub[0]]
    rst0, rs0 = pl.pallas_call(
        _tc1_body,
        grid=(nblk,),
        in_specs=[_rowspec(NB, nin), _rowspec(NB, nfreq), _degspec(NB)]
                 + [_wspec(w.shape) for w in w1],
        out_specs=[_rowspec(NB, nhid),
                   pl.BlockSpec((2, NB, hw), lambda i: (0, i, 0))],
        out_shape=[jax.ShapeDtypeStruct((N, nhid), f32),
                   jax.ShapeDtypeStruct((2, N, hw), f32)],
        compiler_params=cparams,
    )(T, tf2, deg_tc, *w1)

    ayr0 = _lap_call(rs0, src_g, dst_g, acc_rows)
    ay0_tc = ayr0[:, :N, :].reshape(2, nblk, NB, hw)

    w2 = [WoT, bo, wh, bh, wt, bt, WsT, bs, C1m, Kd[0:1], RSaT[0], RSbT[0],
          rsb[0], HEaT[1], HEbT[1], heb[1], K1T[1], k1b[1], K2T[1], k2b[1],
          KUT[1], kub[1]]
    rst1, rs1 = pl.pallas_call(
        _tc2_body,
        grid=(nblk,),
        in_specs=[_rowspec(NB, nin), _rowspec(NB, nfreq), _degspec(NB),
                  _rowspec(NB, nhid), _ayspec(NB, hw)]
                 + [_wspec(w.shape) for w in w2],
        out_specs=[_rowspec(NB, nhid),
                   pl.BlockSpec((2, NB, hw), lambda i: (0, i, 0))],
        out_shape=[jax.ShapeDtypeStruct((N, nhid), f32),
                   jax.ShapeDtypeStruct((2, N, hw), f32)],
        compiler_params=cparams,
    )(T, tf2, deg_tc, rst0, ay0_tc, *w2)

    ayr1 = _lap_call(rs1, src_g, dst_g, acc_rows)
    ay1_tc = ayr1[:, :N, :].reshape(2, nblk, NB, hw)

    nout = W_close.shape[0]
    w3 = [Kd[1:2], RSaT[1], RSbT[1], rsb[1], WcT, bc]
    out = pl.pallas_call(
        _tc3_body,
        grid=(nblk,),
        in_specs=[_rowspec(NB, nin), _degspec(NB), _rowspec(NB, nhid),
                  _ayspec(NB, hw)] + [_wspec(w.shape) for w in w3],
        out_specs=_rowspec(NB, nout),
        out_shape=jax.ShapeDtypeStruct((N, nout), f32),
        compiler_params=cparams,
    )(T, deg_tc, rst1, ay1_tc, *w3)
    return out


# speedup vs baseline: 11.0090x; 1.2897x over previous
import functools

import jax
import jax.numpy as jnp
from jax import lax
from jax.experimental import pallas as pl
from jax.experimental.pallas import tpu as pltpu
from jax.experimental.pallas import tpu_sc as plsc

NC = 2
NS = 16
L = 16

H = 0.1
S0 = float((1.0 + 1e-5) ** -0.5)

STEP = 128
KCH = 8


def _round_up(x, m):
    return (x + m - 1) // m * m


def _fill(buf, val):
    v = jnp.full((L,), val, jnp.float32)

    def body(i, _):
        buf[i, pl.ds(0, L)] = v
        buf[i, pl.ds(L, L)] = v
        return 0
    lax.fori_loop(0, buf.shape[0], body, 0)


def _deg_kernel(stripe, nsteps, src_i, deg_out, acc, idxb, onesb, sem0):
    c = lax.axis_index("c")
    s = lax.axis_index("s")
    _fill(onesb, 0.0)
    for z in range(stripe // 128):
        pltpu.sync_copy(onesb, acc.at[pl.ds(s * stripe + z * 128, 128)])
    _fill(onesb, 1.0)
    plsc.subcore_barrier()

    ebase = (c * NS + s) * nsteps * STEP

    def chunk(g, _):
        pltpu.sync_copy(src_i.at[pl.ds(ebase + g * (KCH * STEP), KCH * STEP)],
                        idxb)
        cps = [pltpu.async_copy(onesb, acc.at[idxb.at[pl.ds(j * STEP, STEP)]],
                                sem0, add=True) for j in range(KCH)]
        for cp in cps:
            cp.wait()
        return 0
    lax.fori_loop(0, nsteps // KCH, chunk, 0)

    plsc.subcore_barrier()
    r0 = s * stripe

    def cp(z, _):
        pltpu.sync_copy(acc.at[pl.ds(r0 + z * 128, 128)], onesb)
        pltpu.sync_copy(onesb, deg_out.at[c, pl.ds(r0 + z * 128, 128)])
        return 0
    lax.fori_loop(0, stripe // 128, cp, 0)


@functools.partial(jax.jit, static_argnums=(1,))
def _deg_call(src_i, nrows):
    stripe = nrows // NS
    nsteps = src_i.shape[0] // (NC * NS * STEP)
    mesh = plsc.VectorSubcoreMesh(core_axis_name="c", subcore_axis_name="s",
                                  num_cores=NC, num_subcores=NS)
    kern = pl.kernel(
        functools.partial(_deg_kernel, stripe, nsteps),
        out_type=jax.ShapeDtypeStruct((NC, nrows, 32), jnp.float32),
        mesh=mesh,
        scratch_types=[
            pltpu.VMEM_SHARED((nrows, 32), jnp.float32),
            pltpu.VMEM((KCH * STEP,), jnp.int32),
            pltpu.VMEM((128, 32), jnp.float32),
            pltpu.SemaphoreType.DMA,
        ],
        compiler_params=pltpu.CompilerParams(use_tc_tiling_on_sc=False),
    )
    return kern(src_i)


def _lap_kernel(stripe, nsteps, table, src_i, dst_i, out, acc, srcb, dstb,
                rows0, rows1, sem0, sem1):
    c = lax.axis_index("c")
    s = lax.axis_index("s")

    _fill(rows0, 0.0)
    for z in range(stripe // 128):
        pltpu.sync_copy(rows0, acc.at[pl.ds(s * stripe + z * 128, 128)])
    plsc.subcore_barrier()

    ebase = s * nsteps * STEP
    rows = (rows0, rows1)
    sems = (sem0, sem1)

    def chunk(g, _):
        coff = ebase + g * (KCH * STEP)
        pltpu.sync_copy(src_i.at[pl.ds(coff, KCH * STEP)], srcb)
        pltpu.sync_copy(dst_i.at[pl.ds(coff, KCH * STEP)], dstb)
        cp_prev = pltpu.async_copy(
            table.at[c].at[srcb.at[pl.ds(0, STEP)]], rows0, sem0)
        for j in range(KCH):
            if j + 1 < KCH:
                cp_next = pltpu.async_copy(
                    table.at[c].at[srcb.at[pl.ds((j + 1) * STEP, STEP)]],
                    rows[(j + 1) % 2], sems[(j + 1) % 2])
            cp_prev.wait()
            pltpu.sync_copy(rows[j % 2],
                            acc.at[dstb.at[pl.ds(j * STEP, STEP)]], add=True)
            if j + 1 < KCH:
                cp_prev = cp_next
        return 0
    lax.fori_loop(0, nsteps // KCH, chunk, 0)

    plsc.subcore_barrier()
    r0 = s * stripe

    def cp(z, _):
        pltpu.sync_copy(acc.at[pl.ds(r0 + z * 128, 128)], rows1)
        pltpu.sync_copy(rows1, out.at[c, pl.ds(r0 + z * 128, 128)])
        return 0
    lax.fori_loop(0, stripe // 128, cp, 0)


@functools.partial(jax.jit, static_argnums=(3,))
def _lap_call(table, src_i, dst_i, acc_rows):
    stripe = acc_rows // NS
    hw = table.shape[2]
    nsteps = src_i.shape[0] // (NS * STEP)
    mesh = plsc.VectorSubcoreMesh(core_axis_name="c", subcore_axis_name="s",
                                  num_cores=NC, num_subcores=NS)
    kern = pl.kernel(
        functools.partial(_lap_kernel, stripe, nsteps),
        out_type=jax.ShapeDtypeStruct((NC, acc_rows, hw), jnp.float32),
        mesh=mesh,
        scratch_types=[
            pltpu.VMEM_SHARED((acc_rows, hw), jnp.float32),
            pltpu.VMEM((KCH * STEP,), jnp.int32),
            pltpu.VMEM((KCH * STEP,), jnp.int32),
            pltpu.VMEM((128, hw), jnp.float32),
            pltpu.VMEM((128, hw), jnp.float32),
            pltpu.SemaphoreType.DMA,
            pltpu.SemaphoreType.DMA,
        ],
        compiler_params=pltpu.CompilerParams(use_tc_tiling_on_sc=False),
    )
    return kern(table, src_i, dst_i)


def _relu_bn(x):
    return jnp.maximum(x * S0, 0.0)


def _dinv_of(deg_r):
    deg = deg_r[0, 0] + deg_r[1, 0]
    return jnp.where(deg > 0, lax.rsqrt(jnp.maximum(deg, 1.0)), 0.0)


def _prologue(Tb, tfb, WoT, bo, wh, bh, wt, bt, WsT, bs):
    Thist = _relu_bn(jnp.dot(Tb, WoT, preferred_element_type=jnp.float32) + bo)
    Tst = _relu_bn(Tb[:, -1:] * wh + bh)
    pre = jnp.dot(tfb, wt, preferred_element_type=jnp.float32) + bt
    te = pre * jax.nn.sigmoid(pre)
    Tst = Tst + te * WsT + bs
    return Thist, Tst


def _reaction(Thist, comb, HEaT, HEbT, heb, K1T, k1b, K2T, k2b, KUT, kub):
    Th = _relu_bn(jnp.dot(Thist, HEaT, preferred_element_type=jnp.float32)
                  + jnp.dot(comb, HEbT, preferred_element_type=jnp.float32) + heb)
    dT = (jnp.dot(Th, K1T, preferred_element_type=jnp.float32) + k1b
          + jnp.dot(Thist, KUT, preferred_element_type=jnp.float32) + kub
          + comb * jnp.clip(jnp.dot(Th, K2T, preferred_element_type=jnp.float32)
                            + k2b, -1.0, 1.0))
    return _relu_bn(comb + H * dT)


def _diffusion(Rst, ay_r, dinv, Kd):
    ay = jnp.concatenate([ay_r[0, 0], ay_r[1, 0]], axis=-1) * dinv
    return Rst - H * Kd * (Rst - ay)


def _tc1_body(T_r, tf_r, deg_r, WoT_r, bo_r, wh_r, bh_r, wt_r, bt_r, WsT_r,
              bs_r, C0_r, HEaT_r, HEbT_r, heb_r, K1T_r, k1b_r, K2T_r, k2b_r,
              KUT_r, kub_r, rst_out, rs2_out):
    Tb = T_r[...]
    Thist, Tst = _prologue(Tb, tf_r[...], WoT_r[...], bo_r[...], wh_r[...],
                           bh_r[...], wt_r[...], bt_r[...], WsT_r[...],
                           bs_r[...])
    comb = Tst * C0_r[:, 0:1]
    Rst = _reaction(Thist, comb, HEaT_r[...], HEbT_r[...], heb_r[...],
                    K1T_r[...], k1b_r[...], K2T_r[...], k2b_r[...],
                    KUT_r[...], kub_r[...])
    rst_out[...] = Rst
    dinv = _dinv_of(deg_r)
    Rs = Rst * dinv
    rs2_out[0] = Rs[:, :32]
    rs2_out[1] = Rs[:, 32:]


def _tc2_body(T_r, tf_r, deg_r, rst0_r, ay_r, WoT_r, bo_r, wh_r, bh_r, wt_r,
              bt_r, WsT_r, bs_r, C1_r, Kd0_r, RSaT_r, RSbT_r, rsb_r, HEaT_r,
              HEbT_r, heb_r, K1T_r, k1b_r, K2T_r, k2b_r, KUT_r, kub_r,
              rst_out, rs2_out):
    Tb = T_r[...]
    dinv = _dinv_of(deg_r)
    Dst = _diffusion(rst0_r[...], ay_r, dinv, Kd0_r[...])
    Tnew1 = _relu_bn(jnp.dot(Tb, RSaT_r[...], preferred_element_type=jnp.float32)
                     + jnp.dot(Dst, RSbT_r[...], preferred_element_type=jnp.float32)
                     + rsb_r[...])
    Thist, Tst = _prologue(Tb, tf_r[...], WoT_r[...], bo_r[...], wh_r[...],
                           bh_r[...], wt_r[...], bt_r[...], WsT_r[...],
                           bs_r[...])
    comb = Tst * C1_r[:, 0:1] + Tnew1 * C1_r[:, 1:2]
    Rst = _reaction(Thist, comb, HEaT_r[...], HEbT_r[...], heb_r[...],
                    K1T_r[...], k1b_r[...], K2T_r[...], k2b_r[...],
                    KUT_r[...], kub_r[...])
    rst_out[...] = Rst
    Rs = Rst * dinv
    rs2_out[0] = Rs[:, :32]
    rs2_out[1] = Rs[:, 32:]


def _tc3_body(T_r, deg_r, rst1_r, ay_r, Kd1_r, RSaT_r, RSbT_r, rsb_r, WcT_r,
              bc_r, out_r):
    Tb = T_r[...]
    dinv = _dinv_of(deg_r)
    Dst = _diffusion(rst1_r[...], ay_r, dinv, Kd1_r[...])
    Tnew2 = _relu_bn(jnp.dot(Tb, RSaT_r[...], preferred_element_type=jnp.float32)
                     + jnp.dot(Dst, RSbT_r[...], preferred_element_type=jnp.float32)
                     + rsb_r[...])
    out_r[...] = (jnp.dot(Tnew2, WcT_r[...], preferred_element_type=jnp.float32)
                  + bc_r[...])


def _wspec(shape):
    nd = len(shape)
    return pl.BlockSpec(shape, lambda i: (0,) * nd)


def _rowspec(nb, d):
    return pl.BlockSpec((nb, d), lambda i: (i, 0))


def _degspec(nb):
    return pl.BlockSpec((2, 1, nb, 1), lambda i: (0, i, 0, 0))


def _ayspec(nb, hw):
    return pl.BlockSpec((2, 1, nb, hw), lambda i: (0, i, 0, 0))


def kernel(T, time_feature, edge_index, W_openHist, b_openHist, w_hist, b_hist,
           w_time, b_time, W_state, b_state, KR1_W, KR1_b, KR2_W, KR2_b,
           KRU0_W, KRU0_b, Kappa, HE_W, HE_b, RS_W, RS_b, C0, C1, W_close,
           b_close):
    N = T.shape[0]
    E = edge_index.shape[1]
    nin = T.shape[1]
    nhid = W_openHist.shape[0]
    hw = nhid // 2

    NB = 1000
    while N % NB:
        NB -= 8
    nblk = N // NB

    e_pad = _round_up(E, NC * NS * STEP * KCH)
    acc_rows = _round_up(N + 1, NS * 128)
    stripe = acc_rows // NS

    src = edge_index[0]
    dst = edge_index[1]
    pad = e_pad - E
    src_g = jnp.concatenate([src, jnp.zeros((pad,), jnp.int32)])
    dst_g = jnp.concatenate([dst, jnp.full((pad,), N, jnp.int32)])
    src_d = jnp.concatenate([src, jnp.full((pad,), N, jnp.int32)])

    f32 = jnp.float32
    WoT = W_openHist.T
    bo = b_openHist[None]
    wh = w_hist[None]
    bh = b_hist[None]
    wt = w_time[:, None]
    bt = b_time[None]
    WsT = W_state.T
    bs = b_state[None]
    C0m = C0[None].astype(f32)
    C1m = C1[None].astype(f32)
    Kd = jnp.clip(Kappa, 0.0, 1.0)
    HEaT = [(HE_W[i][:, :nhid] + HE_W[i][:, 2 * nhid:]).T for i in range(2)]
    HEbT = [HE_W[i][:, nhid:2 * nhid].T for i in range(2)]
    heb = [HE_b[i][None] for i in range(2)]
    K1T = [KR1_W[i].T for i in range(2)]
    k1b = [KR1_b[i][None] for i in range(2)]
    K2T = [KR2_W[i].T for i in range(2)]
    k2b = [KR2_b[i][None] for i in range(2)]
    KUT = [KRU0_W[i].T for i in range(2)]
    kub = [KRU0_b[i][None] for i in range(2)]
    RSaT = [RS_W[i][:, :nin].T for i in range(2)]
    RSbT = [RS_W[i][:, nin:].T for i in range(2)]
    rsb = [RS_b[i][None] for i in range(2)]
    WcT = W_close.T
    bc = b_close[None]

    tf2 = time_feature.reshape(N, -1)
    nfreq = tf2.shape[1]

    deg_p = _deg_call(src_d, acc_rows)
    deg_tc = deg_p[:, :N, 0:1].reshape(2, nblk, NB, 1)

    cparams = pltpu.CompilerParams(dimension_semantics=("arbitrary",))

    w1 = [WoT, bo, wh, bh, wt, bt, WsT, bs, C0m, HEaT[0], HEbT[0], heb[0],
          K1T[0], k1b[0], K2T[0], k2b[0], KUT[0], kub[0]]
    rst0, rs0 = pl.pallas_call(
        _tc1_body,
        grid=(nblk,),
        in_specs=[_rowspec(NB, nin), _rowspec(NB, nfreq), _degspec(NB)]
                 + [_wspec(w.shape) for w in w1],
        out_specs=[_rowspec(NB, nhid),
                   pl.BlockSpec((2, NB, hw), lambda i: (0, i, 0))],
        out_shape=[jax.ShapeDtypeStruct((N, nhid), f32),
                   jax.ShapeDtypeStruct((2, N, hw), f32)],
        compiler_params=cparams,
    )(T, tf2, deg_tc, *w1)

    ayr0 = _lap_call(rs0, src_g, dst_g, acc_rows)
    ay0_tc = ayr0[:, :N, :].reshape(2, nblk, NB, hw)

    w2 = [WoT, bo, wh, bh, wt, bt, WsT, bs, C1m, Kd[0:1], RSaT[0], RSbT[0],
          rsb[0], HEaT[1], HEbT[1], heb[1], K1T[1], k1b[1], K2T[1], k2b[1],
          KUT[1], kub[1]]
    rst1, rs1 = pl.pallas_call(
        _tc2_body,
        grid=(nblk,),
        in_specs=[_rowspec(NB, nin), _rowspec(NB, nfreq), _degspec(NB),
                  _rowspec(NB, nhid), _ayspec(NB, hw)]
                 + [_wspec(w.shape) for w in w2],
        out_specs=[_rowspec(NB, nhid),
                   pl.BlockSpec((2, NB, hw), lambda i: (0, i, 0))],
        out_shape=[jax.ShapeDtypeStruct((N, nhid), f32),
                   jax.ShapeDtypeStruct((2, N, hw), f32)],
        compiler_params=cparams,
    )(T, tf2, deg_tc, rst0, ay0_tc, *w2)

    ayr1 = _lap_call(rs1, src_g, dst_g, acc_rows)
    ay1_tc = ayr1[:, :N, :].reshape(2, nblk, NB, hw)

    nout = W_close.shape[0]
    w3 = [Kd[1:2], RSaT[1], RSbT[1], rsb[1], WcT, bc]
    out = pl.pallas_call(
        _tc3_body,
        grid=(nblk,),
        in_specs=[_rowspec(NB, nin), _degspec(NB), _rowspec(NB, nhid),
                  _ayspec(NB, hw)] + [_wspec(w.shape) for w in w3],
        out_specs=_rowspec(NB, nout),
        out_shape=jax.ShapeDtypeStruct((N, nout), f32),
        compiler_params=cparams,
    )(T, deg_tc, rst1, ay1_tc, *w3)
    return out


# speedup vs baseline: 13.0054x; 1.1813x over previous
import functools

import jax
import jax.numpy as jnp
from jax import lax
from jax.experimental import pallas as pl
from jax.experimental.pallas import tpu as pltpu
from jax.experimental.pallas import tpu_sc as plsc

NC = 2
NS = 16
L = 16

H = 0.1
S0 = float((1.0 + 1e-5) ** -0.5)

STEP = 128
KCH = 8


def _round_up(x, m):
    return (x + m - 1) // m * m


def _fill(buf, val):
    v = jnp.full((L,), val, jnp.float32)

    def body(i, _):
        buf[i, pl.ds(0, L)] = v
        buf[i, pl.ds(L, L)] = v
        return 0
    lax.fori_loop(0, buf.shape[0], body, 0)


def _deg_kernel(stripe, nsteps, src_i, deg_out, acc, idxb, onesb, sem0):
    c = lax.axis_index("c")
    s = lax.axis_index("s")
    _fill(onesb, 0.0)
    for z in range(stripe // 128):
        pltpu.sync_copy(onesb, acc.at[pl.ds(s * stripe + z * 128, 128)])
    _fill(onesb, 1.0)
    plsc.subcore_barrier()

    ebase = (c * NS + s) * nsteps * STEP

    def chunk(g, _):
        pltpu.sync_copy(src_i.at[pl.ds(ebase + g * (KCH * STEP), KCH * STEP)],
                        idxb)
        cps = [pltpu.async_copy(onesb, acc.at[idxb.at[pl.ds(j * STEP, STEP)]],
                                sem0, add=True) for j in range(KCH)]
        for cp in cps:
            cp.wait()
        return 0
    lax.fori_loop(0, nsteps // KCH, chunk, 0)

    plsc.subcore_barrier()
    r0 = s * stripe

    def cp(z, _):
        pltpu.sync_copy(acc.at[pl.ds(r0 + z * 128, 128)], onesb)
        pltpu.sync_copy(onesb, deg_out.at[c, pl.ds(r0 + z * 128, 128)])
        return 0
    lax.fori_loop(0, stripe // 128, cp, 0)


@functools.partial(jax.jit, static_argnums=(1,))
def _deg_call(src_i, nrows):
    stripe = nrows // NS
    nsteps = src_i.shape[0] // (NC * NS * STEP)
    mesh = plsc.VectorSubcoreMesh(core_axis_name="c", subcore_axis_name="s",
                                  num_cores=NC, num_subcores=NS)
    kern = pl.kernel(
        functools.partial(_deg_kernel, stripe, nsteps),
        out_type=jax.ShapeDtypeStruct((NC, nrows, 32), jnp.float32),
        mesh=mesh,
        scratch_types=[
            pltpu.VMEM_SHARED((nrows, 32), jnp.float32),
            pltpu.VMEM((KCH * STEP,), jnp.int32),
            pltpu.VMEM((128, 32), jnp.float32),
            pltpu.SemaphoreType.DMA,
        ],
        compiler_params=pltpu.CompilerParams(use_tc_tiling_on_sc=False),
    )
    return kern(src_i)


def _lap_kernel(stripe, nsteps, table, src_i, dst_i, out, acc, srcb, dstb,
                rows0, rows1, rows2, rows3, sem0, sem1, sem2, sem3, semis,
                semid):
    c = lax.axis_index("c")
    s = lax.axis_index("s")

    _fill(rows0, 0.0)
    for z in range(stripe // 128):
        pltpu.sync_copy(rows0, acc.at[pl.ds(s * stripe + z * 128, 128)])
    plsc.subcore_barrier()

    ebase = s * nsteps * STEP
    nchunks = nsteps // KCH
    CL = KCH * STEP
    rows = (rows0, rows1, rows2, rows3)
    gsem = (sem0, sem1, sem2, sem3)
    R = len(rows)

    pltpu.async_copy(src_i.at[pl.ds(ebase, CL)], srcb.at[pl.ds(0, CL)], semis)
    pltpu.async_copy(dst_i.at[pl.ds(ebase, CL)], dstb.at[pl.ds(0, CL)], semid)

    def chunk(g, _):
        coff = ebase + g * CL
        goff = lax.rem(g, 2) * CL
        noff = CL - goff
        pltpu.make_async_copy(src_i.at[pl.ds(coff, CL)],
                              srcb.at[pl.ds(goff, CL)], semis).wait()
        pltpu.make_async_copy(dst_i.at[pl.ds(coff, CL)],
                              dstb.at[pl.ds(goff, CL)], semid).wait()

        @pl.when(g + 1 < nchunks)
        def _():
            pltpu.async_copy(src_i.at[pl.ds(coff + CL, CL)],
                             srcb.at[pl.ds(noff, CL)], semis)
            pltpu.async_copy(dst_i.at[pl.ds(coff + CL, CL)],
                             dstb.at[pl.ds(noff, CL)], semid)

        def gather(j, buf, sem):
            return pltpu.async_copy(
                table.at[c].at[srcb.at[pl.ds(goff + j * STEP, STEP)]],
                buf, sem)

        cp = [None] * KCH
        for j in range(R):
            cp[j] = gather(j, rows[j], gsem[j])
        for j in range(KCH):
            cp[j].wait()
            pltpu.sync_copy(rows[j % R],
                            acc.at[dstb.at[pl.ds(goff + j * STEP, STEP)]],
                            add=True)
            nj = j + R
            if nj < KCH:
                cp[nj] = gather(nj, rows[j % R], gsem[j % R])
        return 0
    lax.fori_loop(0, nchunks, chunk, 0)

    plsc.subcore_barrier()
    r0 = s * stripe

    def cp(z, _):
        pltpu.sync_copy(acc.at[pl.ds(r0 + z * 128, 128)], rows1)
        pltpu.sync_copy(rows1, out.at[c, pl.ds(r0 + z * 128, 128)])
        return 0
    lax.fori_loop(0, stripe // 128, cp, 0)


@functools.partial(jax.jit, static_argnums=(3,))
def _lap_call(table, src_i, dst_i, acc_rows):
    stripe = acc_rows // NS
    hw = table.shape[2]
    nsteps = src_i.shape[0] // (NS * STEP)
    mesh = plsc.VectorSubcoreMesh(core_axis_name="c", subcore_axis_name="s",
                                  num_cores=NC, num_subcores=NS)
    kern = pl.kernel(
        functools.partial(_lap_kernel, stripe, nsteps),
        out_type=jax.ShapeDtypeStruct((NC, acc_rows, hw), jnp.float32),
        mesh=mesh,
        scratch_types=[
            pltpu.VMEM_SHARED((acc_rows, hw), jnp.float32),
            pltpu.VMEM((2 * KCH * STEP,), jnp.int32),
            pltpu.VMEM((2 * KCH * STEP,), jnp.int32),
            pltpu.VMEM((128, hw), jnp.float32),
            pltpu.VMEM((128, hw), jnp.float32),
            pltpu.VMEM((128, hw), jnp.float32),
            pltpu.VMEM((128, hw), jnp.float32),
            pltpu.SemaphoreType.DMA,
            pltpu.SemaphoreType.DMA,
            pltpu.SemaphoreType.DMA,
            pltpu.SemaphoreType.DMA,
            pltpu.SemaphoreType.DMA,
            pltpu.SemaphoreType.DMA,
        ],
        compiler_params=pltpu.CompilerParams(use_tc_tiling_on_sc=False),
    )
    return kern(table, src_i, dst_i)


def _relu_bn(x):
    return jnp.maximum(x * S0, 0.0)


def _dinv_of(deg_r):
    deg = deg_r[0][:, 0:1] + deg_r[1][:, 0:1]
    return jnp.where(deg > 0, lax.rsqrt(jnp.maximum(deg, 1.0)), 0.0)


def _prologue(Tb, tfb, WoT, bo, wh, bh, wt, bt, WsT, bs):
    Thist = _relu_bn(jnp.dot(Tb, WoT, preferred_element_type=jnp.float32) + bo)
    Tst = _relu_bn(Tb[:, -1:] * wh + bh)
    pre = jnp.dot(tfb, wt, preferred_element_type=jnp.float32) + bt
    te = pre * jax.nn.sigmoid(pre)
    Tst = Tst + te * WsT + bs
    return Thist, Tst


def _reaction(Thist, comb, HEaT, HEbT, heb, K1T, k1b, K2T, k2b, KUT, kub):
    Th = _relu_bn(jnp.dot(Thist, HEaT, preferred_element_type=jnp.float32)
                  + jnp.dot(comb, HEbT, preferred_element_type=jnp.float32) + heb)
    dT = (jnp.dot(Th, K1T, preferred_element_type=jnp.float32) + k1b
          + jnp.dot(Thist, KUT, preferred_element_type=jnp.float32) + kub
          + comb * jnp.clip(jnp.dot(Th, K2T, preferred_element_type=jnp.float32)
                            + k2b, -1.0, 1.0))
    return _relu_bn(comb + H * dT)


def _diffusion(Rst, ay_r, dinv, Kd):
    ay = jnp.concatenate([ay_r[0], ay_r[1]], axis=-1) * dinv
    return Rst - H * Kd * (Rst - ay)


def _tc1_body(T_r, tf_r, deg_r, WoT_r, bo_r, wh_r, bh_r, wt_r, bt_r, WsT_r,
              bs_r, C0_r, HEaT_r, HEbT_r, heb_r, K1T_r, k1b_r, K2T_r, k2b_r,
              KUT_r, kub_r, rst_out, rs2_out):
    Tb = T_r[...]
    Thist, Tst = _prologue(Tb, tf_r[...], WoT_r[...], bo_r[...], wh_r[...],
                           bh_r[...], wt_r[...], bt_r[...], WsT_r[...],
                           bs_r[...])
    comb = Tst * C0_r[:, 0:1]
    Rst = _reaction(Thist, comb, HEaT_r[...], HEbT_r[...], heb_r[...],
                    K1T_r[...], k1b_r[...], K2T_r[...], k2b_r[...],
                    KUT_r[...], kub_r[...])
    rst_out[...] = Rst
    dinv = _dinv_of(deg_r)
    Rs = Rst * dinv
    rs2_out[0] = Rs[:, :32]
    rs2_out[1] = Rs[:, 32:]


def _tc2_body(T_r, tf_r, deg_r, rst0_r, ay_r, WoT_r, bo_r, wh_r, bh_r, wt_r,
              bt_r, WsT_r, bs_r, C1_r, Kd0_r, RSaT_r, RSbT_r, rsb_r, HEaT_r,
              HEbT_r, heb_r, K1T_r, k1b_r, K2T_r, k2b_r, KUT_r, kub_r,
              rst_out, rs2_out):
    Tb = T_r[...]
    dinv = _dinv_of(deg_r)
    Dst = _diffusion(rst0_r[...], ay_r, dinv, Kd0_r[...])
    Tnew1 = _relu_bn(jnp.dot(Tb, RSaT_r[...], preferred_element_type=jnp.float32)
                     + jnp.dot(Dst, RSbT_r[...], preferred_element_type=jnp.float32)
                     + rsb_r[...])
    Thist, Tst = _prologue(Tb, tf_r[...], WoT_r[...], bo_r[...], wh_r[...],
                           bh_r[...], wt_r[...], bt_r[...], WsT_r[...],
                           bs_r[...])
    comb = Tst * C1_r[:, 0:1] + Tnew1 * C1_r[:, 1:2]
    Rst = _reaction(Thist, comb, HEaT_r[...], HEbT_r[...], heb_r[...],
                    K1T_r[...], k1b_r[...], K2T_r[...], k2b_r[...],
                    KUT_r[...], kub_r[...])
    rst_out[...] = Rst
    Rs = Rst * dinv
    rs2_out[0] = Rs[:, :32]
    rs2_out[1] = Rs[:, 32:]


def _tc3_body(T_r, deg_r, rst1_r, ay_r, Kd1_r, RSaT_r, RSbT_r, rsb_r, WcT_r,
              bc_r, out_r):
    Tb = T_r[...]
    dinv = _dinv_of(deg_r)
    Dst = _diffusion(rst1_r[...], ay_r, dinv, Kd1_r[...])
    Tnew2 = _relu_bn(jnp.dot(Tb, RSaT_r[...], preferred_element_type=jnp.float32)
                     + jnp.dot(Dst, RSbT_r[...], preferred_element_type=jnp.float32)
                     + rsb_r[...])
    out_r[...] = (jnp.dot(Tnew2, WcT_r[...], preferred_element_type=jnp.float32)
                  + bc_r[...])


def _wspec(shape):
    nd = len(shape)
    return pl.BlockSpec(shape, lambda i: (0,) * nd)


def _rowspec(nb, d):
    return pl.BlockSpec((nb, d), lambda i: (i, 0))


def _degspec(nb):
    return pl.BlockSpec((2, nb, 32), lambda i: (0, i, 0))


def _ayspec(nb, hw):
    return pl.BlockSpec((2, nb, hw), lambda i: (0, i, 0))


def kernel(T, time_feature, edge_index, W_openHist, b_openHist, w_hist, b_hist,
           w_time, b_time, W_state, b_state, KR1_W, KR1_b, KR2_W, KR2_b,
           KRU0_W, KRU0_b, Kappa, HE_W, HE_b, RS_W, RS_b, C0, C1, W_close,
           b_close):
    N = T.shape[0]
    E = edge_index.shape[1]
    nin = T.shape[1]
    nhid = W_openHist.shape[0]
    hw = nhid // 2

    NB = 1000
    while N % NB:
        NB -= 8
    nblk = N // NB

    e_pad = _round_up(E, NC * NS * STEP * KCH)
    acc_rows = _round_up(N + 1, NS * 128)
    stripe = acc_rows // NS

    src = edge_index[0]
    dst = edge_index[1]
    pad = e_pad - E
    src_g = jnp.concatenate([src, jnp.zeros((pad,), jnp.int32)])
    dst_g = jnp.concatenate([dst, jnp.full((pad,), N, jnp.int32)])
    src_d = jnp.concatenate([src, jnp.full((pad,), N, jnp.int32)])

    f32 = jnp.float32
    WoT = W_openHist.T
    bo = b_openHist[None]
    wh = w_hist[None]
    bh = b_hist[None]
    wt = w_time[:, None]
    bt = b_time[None]
    WsT = W_state.T
    bs = b_state[None]
    C0m = C0[None].astype(f32)
    C1m = C1[None].astype(f32)
    Kd = jnp.clip(Kappa, 0.0, 1.0)
    HEaT = [(HE_W[i][:, :nhid] + HE_W[i][:, 2 * nhid:]).T for i in range(2)]
    HEbT = [HE_W[i][:, nhid:2 * nhid].T for i in range(2)]
    heb = [HE_b[i][None] for i in range(2)]
    K1T = [KR1_W[i].T for i in range(2)]
    k1b = [KR1_b[i][None] for i in range(2)]
    K2T = [KR2_W[i].T for i in range(2)]
    k2b = [KR2_b[i][None] for i in range(2)]
    KUT = [KRU0_W[i].T for i in range(2)]
    kub = [KRU0_b[i][None] for i in range(2)]
    RSaT = [RS_W[i][:, :nin].T for i in range(2)]
    RSbT = [RS_W[i][:, nin:].T for i in range(2)]
    rsb = [RS_b[i][None] for i in range(2)]
    WcT = W_close.T
    bc = b_close[None]

    tf2 = time_feature.reshape(N, -1)
    nfreq = tf2.shape[1]

    deg_tc = _deg_call(src_d, acc_rows)

    cparams = pltpu.CompilerParams(dimension_semantics=("arbitrary",))

    w1 = [WoT, bo, wh, bh, wt, bt, WsT, bs, C0m, HEaT[0], HEbT[0], heb[0],
          K1T[0], k1b[0], K2T[0], k2b[0], KUT[0], kub[0]]
    rst0, rs0 = pl.pallas_call(
        _tc1_body,
        grid=(nblk,),
        in_specs=[_rowspec(NB, nin), _rowspec(NB, nfreq), _degspec(NB)]
                 + [_wspec(w.shape) for w in w1],
        out_specs=[_rowspec(NB, nhid),
                   pl.BlockSpec((2, NB, hw), lambda i: (0, i, 0))],
        out_shape=[jax.ShapeDtypeStruct((N, nhid), f32),
                   jax.ShapeDtypeStruct((2, N, hw), f32)],
        compiler_params=cparams,
    )(T, tf2, deg_tc, *w1)

    ay0_tc = _lap_call(rs0, src_g, dst_g, acc_rows)

    w2 = [WoT, bo, wh, bh, wt, bt, WsT, bs, C1m, Kd[0:1], RSaT[0], RSbT[0],
          rsb[0], HEaT[1], HEbT[1], heb[1], K1T[1], k1b[1], K2T[1], k2b[1],
          KUT[1], kub[1]]
    rst1, rs1 = pl.pallas_call(
        _tc2_body,
        grid=(nblk,),
        in_specs=[_rowspec(NB, nin), _rowspec(NB, nfreq), _degspec(NB),
                  _rowspec(NB, nhid), _ayspec(NB, hw)]
                 + [_wspec(w.shape) for w in w2],
        out_specs=[_rowspec(NB, nhid),
                   pl.BlockSpec((2, NB, hw), lambda i: (0, i, 0))],
        out_shape=[jax.ShapeDtypeStruct((N, nhid), f32),
                   jax.ShapeDtypeStruct((2, N, hw), f32)],
        compiler_params=cparams,
    )(T, tf2, deg_tc, rst0, ay0_tc, *w2)

    ay1_tc = _lap_call(rs1, src_g, dst_g, acc_rows)

    nout = W_close.shape[0]
    w3 = [Kd[1:2], RSaT[1], RSbT[1], rsb[1], WcT, bc]
    out = pl.pallas_call(
        _tc3_body,
        grid=(nblk,),
        in_specs=[_rowspec(NB, nin), _degspec(NB), _rowspec(NB, nhid),
                  _ayspec(NB, hw)] + [_wspec(w.shape) for w in w3],
        out_specs=_rowspec(NB, nout),
        out_shape=jax.ShapeDtypeStruct((N, nout), f32),
        compiler_params=cparams,
    )(T, deg_tc, rst1, ay1_tc, *w3)
    return out


# speedup vs baseline: 15.3972x; 1.1839x over previous
import functools

import jax
import jax.numpy as jnp
from jax import lax
from jax.experimental import pallas as pl
from jax.experimental.pallas import tpu as pltpu
from jax.experimental.pallas import tpu_sc as plsc

NC = 2
NS = 16
L = 16

H = 0.1
S0 = float((1.0 + 1e-5) ** -0.5)

STEP = 128
KCH = 8


def _round_up(x, m):
    return (x + m - 1) // m * m


def _fill(buf, val):
    lanes = 32 if buf.dtype == jnp.bfloat16 else L
    v = jnp.full((lanes,), val, buf.dtype)
    ncol = buf.shape[1]

    def body(i, _):
        for k in range(ncol // lanes):
            buf[i, pl.ds(k * lanes, lanes)] = v
        return 0
    lax.fori_loop(0, buf.shape[0], body, 0)


def _deg_kernel(stripe, nsteps, src_i, deg_out, acc, idxb, onesb, sem0):
    c = lax.axis_index("c")
    s = lax.axis_index("s")
    _fill(onesb, 0.0)
    for z in range(stripe // 128):
        pltpu.sync_copy(onesb, acc.at[pl.ds(s * stripe + z * 128, 128)])
    _fill(onesb, 1.0)
    plsc.subcore_barrier()

    ebase = (c * NS + s) * nsteps * STEP

    def chunk(g, _):
        pltpu.sync_copy(src_i.at[pl.ds(ebase + g * (KCH * STEP), KCH * STEP)],
                        idxb)
        cps = [pltpu.async_copy(onesb, acc.at[idxb.at[pl.ds(j * STEP, STEP)]],
                                sem0, add=True) for j in range(KCH)]
        for cp in cps:
            cp.wait()
        return 0
    lax.fori_loop(0, nsteps // KCH, chunk, 0)

    plsc.subcore_barrier()
    r0 = s * stripe

    def cp(z, _):
        pltpu.sync_copy(acc.at[pl.ds(r0 + z * 128, 128)], onesb)
        pltpu.sync_copy(onesb, deg_out.at[c, pl.ds(r0 + z * 128, 128)])
        return 0
    lax.fori_loop(0, stripe // 128, cp, 0)


@functools.partial(jax.jit, static_argnums=(1,))
def _deg_call(src_i, nrows):
    stripe = nrows // NS
    nsteps = src_i.shape[0] // (NC * NS * STEP)
    mesh = plsc.VectorSubcoreMesh(core_axis_name="c", subcore_axis_name="s",
                                  num_cores=NC, num_subcores=NS)
    kern = pl.kernel(
        functools.partial(_deg_kernel, stripe, nsteps),
        out_type=jax.ShapeDtypeStruct((NC, nrows, 32), jnp.float32),
        mesh=mesh,
        scratch_types=[
            pltpu.VMEM_SHARED((nrows, 32), jnp.float32),
            pltpu.VMEM((KCH * STEP,), jnp.int32),
            pltpu.VMEM((128, 32), jnp.float32),
            pltpu.SemaphoreType.DMA,
        ],
        compiler_params=pltpu.CompilerParams(use_tc_tiling_on_sc=False),
    )
    return kern(src_i)


def _lap_kernel(stripe, nsteps, table, src_i, dst_i, out, acc, srcb, dstb,
                rows0, rows1, rows2, rows3, sem0, sem1, sem2, sem3, semis,
                semid):
    c = lax.axis_index("c")
    s = lax.axis_index("s")

    _fill(rows0, 0.0)
    for z in range(stripe // 128):
        pltpu.sync_copy(rows0, acc.at[pl.ds(s * stripe + z * 128, 128)])
    plsc.subcore_barrier()

    ebase = (c * NS + s) * nsteps * STEP
    nchunks = nsteps // KCH
    CL = KCH * STEP
    rows = (rows0, rows1, rows2, rows3)
    gsem = (sem0, sem1, sem2, sem3)
    R = len(rows)

    pltpu.async_copy(src_i.at[pl.ds(ebase, CL)], srcb.at[pl.ds(0, CL)], semis)
    pltpu.async_copy(dst_i.at[pl.ds(ebase, CL)], dstb.at[pl.ds(0, CL)], semid)

    def chunk(g, _):
        coff = ebase + g * CL
        goff = lax.rem(g, 2) * CL
        noff = CL - goff
        pltpu.make_async_copy(src_i.at[pl.ds(coff, CL)],
                              srcb.at[pl.ds(goff, CL)], semis).wait()
        pltpu.make_async_copy(dst_i.at[pl.ds(coff, CL)],
                              dstb.at[pl.ds(goff, CL)], semid).wait()

        @pl.when(g + 1 < nchunks)
        def _():
            pltpu.async_copy(src_i.at[pl.ds(coff + CL, CL)],
                             srcb.at[pl.ds(noff, CL)], semis)
            pltpu.async_copy(dst_i.at[pl.ds(coff + CL, CL)],
                             dstb.at[pl.ds(noff, CL)], semid)

        def gather(j, buf, sem):
            return pltpu.async_copy(
                table.at[srcb.at[pl.ds(goff + j * STEP, STEP)]], buf, sem)

        cp = [None] * KCH
        for j in range(R):
            cp[j] = gather(j, rows[j], gsem[j])
        for j in range(KCH):
            cp[j].wait()
            pltpu.sync_copy(rows[j % R],
                            acc.at[dstb.at[pl.ds(goff + j * STEP, STEP)]],
                            add=True)
            nj = j + R
            if nj < KCH:
                cp[nj] = gather(nj, rows[j % R], gsem[j % R])
        return 0
    lax.fori_loop(0, nchunks, chunk, 0)

    plsc.subcore_barrier()
    r0 = s * stripe

    def cp(z, _):
        pltpu.sync_copy(acc.at[pl.ds(r0 + z * 128, 128)], rows1)
        pltpu.sync_copy(rows1, out.at[c, pl.ds(r0 + z * 128, 128)])
        return 0
    lax.fori_loop(0, stripe // 128, cp, 0)


@functools.partial(jax.jit, static_argnums=(3,))
def _lap_call(table, src_i, dst_i, acc_rows):
    stripe = acc_rows // NS
    hw = table.shape[1]
    nsteps = src_i.shape[0] // (NC * NS * STEP)
    mesh = plsc.VectorSubcoreMesh(core_axis_name="c", subcore_axis_name="s",
                                  num_cores=NC, num_subcores=NS)
    kern = pl.kernel(
        functools.partial(_lap_kernel, stripe, nsteps),
        out_type=jax.ShapeDtypeStruct((NC, acc_rows, hw), jnp.bfloat16),
        mesh=mesh,
        scratch_types=[
            pltpu.VMEM_SHARED((acc_rows, hw), jnp.bfloat16),
            pltpu.VMEM((2 * KCH * STEP,), jnp.int32),
            pltpu.VMEM((2 * KCH * STEP,), jnp.int32),
            pltpu.VMEM((128, hw), jnp.bfloat16),
            pltpu.VMEM((128, hw), jnp.bfloat16),
            pltpu.VMEM((128, hw), jnp.bfloat16),
            pltpu.VMEM((128, hw), jnp.bfloat16),
            pltpu.SemaphoreType.DMA,
            pltpu.SemaphoreType.DMA,
            pltpu.SemaphoreType.DMA,
            pltpu.SemaphoreType.DMA,
            pltpu.SemaphoreType.DMA,
            pltpu.SemaphoreType.DMA,
        ],
        compiler_params=pltpu.CompilerParams(use_tc_tiling_on_sc=False),
    )
    return kern(table, src_i, dst_i)


def _relu_bn(x):
    return jnp.maximum(x * S0, 0.0)


def _dinv_of(deg_r):
    deg = deg_r[0][:, 0:1] + deg_r[1][:, 0:1]
    return jnp.where(deg > 0, lax.rsqrt(jnp.maximum(deg, 1.0)), 0.0)


def _prologue(Tb, tfb, WoT, bo, wh, bh, wt, bt, WsT, bs):
    Thist = _relu_bn(jnp.dot(Tb, WoT, preferred_element_type=jnp.float32) + bo)
    Tst = _relu_bn(Tb[:, -1:] * wh + bh)
    pre = jnp.dot(tfb, wt, preferred_element_type=jnp.float32) + bt
    te = pre * jax.nn.sigmoid(pre)
    Tst = Tst + te * WsT + bs
    return Thist, Tst


def _reaction(Thist, comb, HEaT, HEbT, heb, K1T, k1b, K2T, k2b, KUT, kub):
    Th = _relu_bn(jnp.dot(Thist, HEaT, preferred_element_type=jnp.float32)
                  + jnp.dot(comb, HEbT, preferred_element_type=jnp.float32) + heb)
    dT = (jnp.dot(Th, K1T, preferred_element_type=jnp.float32) + k1b
          + jnp.dot(Thist, KUT, preferred_element_type=jnp.float32) + kub
          + comb * jnp.clip(jnp.dot(Th, K2T, preferred_element_type=jnp.float32)
                            + k2b, -1.0, 1.0))
    return _relu_bn(comb + H * dT)


def _diffusion(Rst, ay_r, dinv, Kd):
    ay = (ay_r[0] + ay_r[1]).astype(jnp.float32) * dinv
    return Rst - H * Kd * (Rst - ay)


def _tc1_body(T_r, tf_r, deg_r, WoT_r, bo_r, wh_r, bh_r, wt_r, bt_r, WsT_r,
              bs_r, C0_r, HEaT_r, HEbT_r, heb_r, K1T_r, k1b_r, K2T_r, k2b_r,
              KUT_r, kub_r, rst_out, rs2_out):
    Tb = T_r[...]
    Thist, Tst = _prologue(Tb, tf_r[...], WoT_r[...], bo_r[...], wh_r[...],
                           bh_r[...], wt_r[...], bt_r[...], WsT_r[...],
                           bs_r[...])
    comb = Tst * C0_r[:, 0:1]
    Rst = _reaction(Thist, comb, HEaT_r[...], HEbT_r[...], heb_r[...],
                    K1T_r[...], k1b_r[...], K2T_r[...], k2b_r[...],
                    KUT_r[...], kub_r[...])
    rst_out[...] = Rst
    dinv = _dinv_of(deg_r)
    rs2_out[...] = (Rst * dinv).astype(jnp.bfloat16)


def _tc2_body(T_r, tf_r, deg_r, rst0_r, ay_r, WoT_r, bo_r, wh_r, bh_r, wt_r,
              bt_r, WsT_r, bs_r, C1_r, Kd0_r, RSaT_r, RSbT_r, rsb_r, HEaT_r,
              HEbT_r, heb_r, K1T_r, k1b_r, K2T_r, k2b_r, KUT_r, kub_r,
              rst_out, rs2_out):
    Tb = T_r[...]
    dinv = _dinv_of(deg_r)
    Dst = _diffusion(rst0_r[...], ay_r, dinv, Kd0_r[...])
    Tnew1 = _relu_bn(jnp.dot(Tb, RSaT_r[...], preferred_element_type=jnp.float32)
                     + jnp.dot(Dst, RSbT_r[...], preferred_element_type=jnp.float32)
                     + rsb_r[...])
    Thist, Tst = _prologue(Tb, tf_r[...], WoT_r[...], bo_r[...], wh_r[...],
                           bh_r[...], wt_r[...], bt_r[...], WsT_r[...],
                           bs_r[...])
    comb = Tst * C1_r[:, 0:1] + Tnew1 * C1_r[:, 1:2]
    Rst = _reaction(Thist, comb, HEaT_r[...], HEbT_r[...], heb_r[...],
                    K1T_r[...], k1b_r[...], K2T_r[...], k2b_r[...],
                    KUT_r[...], kub_r[...])
    rst_out[...] = Rst
    rs2_out[...] = (Rst * dinv).astype(jnp.bfloat16)


def _tc3_body(T_r, deg_r, rst1_r, ay_r, Kd1_r, RSaT_r, RSbT_r, rsb_r, WcT_r,
              bc_r, out_r):
    Tb = T_r[...]
    dinv = _dinv_of(deg_r)
    Dst = _diffusion(rst1_r[...], ay_r, dinv, Kd1_r[...])
    Tnew2 = _relu_bn(jnp.dot(Tb, RSaT_r[...], preferred_element_type=jnp.float32)
                     + jnp.dot(Dst, RSbT_r[...], preferred_element_type=jnp.float32)
                     + rsb_r[...])
    out_r[...] = (jnp.dot(Tnew2, WcT_r[...], preferred_element_type=jnp.float32)
                  + bc_r[...])


def _wspec(shape):
    nd = len(shape)
    return pl.BlockSpec(shape, lambda i: (0,) * nd)


def _rowspec(nb, d):
    return pl.BlockSpec((nb, d), lambda i: (i, 0))


def _degspec(nb):
    return pl.BlockSpec((2, nb, 32), lambda i: (0, i, 0))


def _ayspec(nb, nhid):
    return pl.BlockSpec((2, nb, nhid), lambda i: (0, i, 0))


def kernel(T, time_feature, edge_index, W_openHist, b_openHist, w_hist, b_hist,
           w_time, b_time, W_state, b_state, KR1_W, KR1_b, KR2_W, KR2_b,
           KRU0_W, KRU0_b, Kappa, HE_W, HE_b, RS_W, RS_b, C0, C1, W_close,
           b_close):
    N = T.shape[0]
    E = edge_index.shape[1]
    nin = T.shape[1]
    nhid = W_openHist.shape[0]
    hw = nhid // 2

    NB = 1000
    while N % NB:
        NB -= 8
    nblk = N // NB

    e_pad = _round_up(E, NC * NS * STEP * KCH)
    acc_rows = _round_up(N + 1, NS * 128)
    stripe = acc_rows // NS

    src = edge_index[0]
    dst = edge_index[1]
    pad = e_pad - E
    src_g = jnp.concatenate([src, jnp.zeros((pad,), jnp.int32)])
    dst_g = jnp.concatenate([dst, jnp.full((pad,), N, jnp.int32)])
    src_d = jnp.concatenate([src, jnp.full((pad,), N, jnp.int32)])

    f32 = jnp.float32
    WoT = W_openHist.T
    bo = b_openHist[None]
    wh = w_hist[None]
    bh = b_hist[None]
    wt = w_time[:, None]
    bt = b_time[None]
    WsT = W_state.T
    bs = b_state[None]
    C0m = C0[None].astype(f32)
    C1m = C1[None].astype(f32)
    Kd = jnp.clip(Kappa, 0.0, 1.0)
    HEaT = [(HE_W[i][:, :nhid] + HE_W[i][:, 2 * nhid:]).T for i in range(2)]
    HEbT = [HE_W[i][:, nhid:2 * nhid].T for i in range(2)]
    heb = [HE_b[i][None] for i in range(2)]
    K1T = [KR1_W[i].T for i in range(2)]
    k1b = [KR1_b[i][None] for i in range(2)]
    K2T = [KR2_W[i].T for i in range(2)]
    k2b = [KR2_b[i][None] for i in range(2)]
    KUT = [KRU0_W[i].T for i in range(2)]
    kub = [KRU0_b[i][None] for i in range(2)]
    RSaT = [RS_W[i][:, :nin].T for i in range(2)]
    RSbT = [RS_W[i][:, nin:].T for i in range(2)]
    rsb = [RS_b[i][None] for i in range(2)]
    WcT = W_close.T
    bc = b_close[None]

    tf2 = time_feature.reshape(N, -1)
    nfreq = tf2.shape[1]

    deg_tc = _deg_call(src_d, acc_rows)

    cparams = pltpu.CompilerParams(dimension_semantics=("arbitrary",))

    w1 = [WoT, bo, wh, bh, wt, bt, WsT, bs, C0m, HEaT[0], HEbT[0], heb[0],
          K1T[0], k1b[0], K2T[0], k2b[0], KUT[0], kub[0]]
    rst0, rs0 = pl.pallas_call(
        _tc1_body,
        grid=(nblk,),
        in_specs=[_rowspec(NB, nin), _rowspec(NB, nfreq), _degspec(NB)]
                 + [_wspec(w.shape) for w in w1],
        out_specs=[_rowspec(NB, nhid), _rowspec(NB, nhid)],
        out_shape=[jax.ShapeDtypeStruct((N, nhid), f32),
                   jax.ShapeDtypeStruct((N, nhid), jnp.bfloat16)],
        compiler_params=cparams,
    )(T, tf2, deg_tc, *w1)

    ay0_tc = _lap_call(rs0, src_g, dst_g, acc_rows)

    w2 = [WoT, bo, wh, bh, wt, bt, WsT, bs, C1m, Kd[0:1], RSaT[0], RSbT[0],
          rsb[0], HEaT[1], HEbT[1], heb[1], K1T[1], k1b[1], K2T[1], k2b[1],
          KUT[1], kub[1]]
    rst1, rs1 = pl.pallas_call(
        _tc2_body,
        grid=(nblk,),
        in_specs=[_rowspec(NB, nin), _rowspec(NB, nfreq), _degspec(NB),
                  _rowspec(NB, nhid), _ayspec(NB, nhid)]
                 + [_wspec(w.shape) for w in w2],
        out_specs=[_rowspec(NB, nhid), _rowspec(NB, nhid)],
        out_shape=[jax.ShapeDtypeStruct((N, nhid), f32),
                   jax.ShapeDtypeStruct((N, nhid), jnp.bfloat16)],
        compiler_params=cparams,
    )(T, tf2, deg_tc, rst0, ay0_tc, *w2)

    ay1_tc = _lap_call(rs1, src_g, dst_g, acc_rows)

    nout = W_close.shape[0]
    w3 = [Kd[1:2], RSaT[1], RSbT[1], rsb[1], WcT, bc]
    out = pl.pallas_call(
        _tc3_body,
        grid=(nblk,),
        in_specs=[_rowspec(NB, nin), _degspec(NB), _rowspec(NB, nhid),
                  _ayspec(NB, nhid)] + [_wspec(w.shape) for w in w3],
        out_specs=_rowspec(NB, nout),
        out_shape=jax.ShapeDtypeStruct((N, nout), f32),
        compiler_params=cparams,
    )(T, deg_tc, rst1, ay1_tc, *w3)
    return out


# speedup vs baseline: 15.4347x; 1.0024x over previous
import functools

import jax
import jax.numpy as jnp
from jax import lax
from jax.experimental import pallas as pl
from jax.experimental.pallas import tpu as pltpu
from jax.experimental.pallas import tpu_sc as plsc

NC = 2
NS = 16
L = 16

H = 0.1
S0 = float((1.0 + 1e-5) ** -0.5)

STEP = 128
KCH = 8


def _round_up(x, m):
    return (x + m - 1) // m * m


def _fill(buf, val):
    lanes = 32 if buf.dtype == jnp.bfloat16 else L
    v = jnp.full((lanes,), val, buf.dtype)
    ncol = buf.shape[1]

    def body(i, _):
        for k in range(ncol // lanes):
            buf[i, pl.ds(k * lanes, lanes)] = v
        return 0
    lax.fori_loop(0, buf.shape[0], body, 0)


def _deg_kernel(stripe, nsteps, src_i, deg_out, acc, idxb, onesb, sem0):
    c = lax.axis_index("c")
    s = lax.axis_index("s")
    _fill(onesb, 0.0)
    for z in range(stripe // 128):
        pltpu.sync_copy(onesb, acc.at[pl.ds(s * stripe + z * 128, 128)])
    _fill(onesb, 1.0)
    plsc.subcore_barrier()

    ebase = (c * NS + s) * nsteps * STEP

    def chunk(g, _):
        pltpu.sync_copy(src_i.at[pl.ds(ebase + g * (KCH * STEP), KCH * STEP)],
                        idxb)
        cps = [pltpu.async_copy(onesb, acc.at[idxb.at[pl.ds(j * STEP, STEP)]],
                                sem0, add=True) for j in range(KCH)]
        for cp in cps:
            cp.wait()
        return 0
    lax.fori_loop(0, nsteps // KCH, chunk, 0)

    plsc.subcore_barrier()
    r0 = s * stripe

    def cp(z, _):
        pltpu.sync_copy(acc.at[pl.ds(r0 + z * 128, 128)], onesb)
        pltpu.sync_copy(onesb, deg_out.at[c, pl.ds(r0 + z * 128, 128)])
        return 0
    lax.fori_loop(0, stripe // 128, cp, 0)


@functools.partial(jax.jit, static_argnums=(1,))
def _deg_call(src_i, nrows):
    stripe = nrows // NS
    nsteps = src_i.shape[0] // (NC * NS * STEP)
    mesh = plsc.VectorSubcoreMesh(core_axis_name="c", subcore_axis_name="s",
                                  num_cores=NC, num_subcores=NS)
    kern = pl.kernel(
        functools.partial(_deg_kernel, stripe, nsteps),
        out_type=jax.ShapeDtypeStruct((NC, nrows, 32), jnp.float32),
        mesh=mesh,
        scratch_types=[
            pltpu.VMEM_SHARED((nrows, 32), jnp.float32),
            pltpu.VMEM((KCH * STEP,), jnp.int32),
            pltpu.VMEM((128, 32), jnp.float32),
            pltpu.SemaphoreType.DMA,
        ],
        compiler_params=pltpu.CompilerParams(use_tc_tiling_on_sc=False),
    )
    return kern(src_i)


def _lap_kernel(stripe, nsteps, table, src_i, dst_i, out, acc, srcb, dstb,
                rows0, rows1, rows2, rows3, rows4, sem0, sem1, sem2, sem3,
                sem4, ssem0, ssem1, semis, semid):
    c = lax.axis_index("c")
    s = lax.axis_index("s")

    _fill(rows0, 0.0)
    for z in range(stripe // 128):
        pltpu.sync_copy(rows0, acc.at[pl.ds(s * stripe + z * 128, 128)])
    plsc.subcore_barrier()

    ebase = (c * NS + s) * nsteps * STEP
    nchunks = nsteps // KCH
    CL = KCH * STEP
    rows = (rows0, rows1, rows2, rows3, rows4)
    gsem = (sem0, sem1, sem2, sem3, sem4)
    ssem = (ssem0, ssem1)
    R = len(rows)

    pltpu.async_copy(src_i.at[pl.ds(ebase, CL)], srcb.at[pl.ds(0, CL)], semis)
    pltpu.async_copy(dst_i.at[pl.ds(ebase, CL)], dstb.at[pl.ds(0, CL)], semid)

    def chunk(g, _):
        coff = ebase + g * CL
        goff = lax.rem(g, 2) * CL
        noff = CL - goff
        pltpu.make_async_copy(src_i.at[pl.ds(coff, CL)],
                              srcb.at[pl.ds(goff, CL)], semis).wait()
        pltpu.make_async_copy(dst_i.at[pl.ds(coff, CL)],
                              dstb.at[pl.ds(goff, CL)], semid).wait()

        @pl.when(g + 1 < nchunks)
        def _():
            pltpu.async_copy(src_i.at[pl.ds(coff + CL, CL)],
                             srcb.at[pl.ds(noff, CL)], semis)
            pltpu.async_copy(dst_i.at[pl.ds(coff + CL, CL)],
                             dstb.at[pl.ds(noff, CL)], semid)

        def gather(j, buf, sem):
            return pltpu.async_copy(
                table.at[srcb.at[pl.ds(goff + j * STEP, STEP)]], buf, sem)

        cp = [None] * KCH
        scp = [None] * KCH
        for j in range(R):
            cp[j] = gather(j, rows[j], gsem[j])
        for j in range(KCH):
            cp[j].wait()
            scp[j] = pltpu.async_copy(
                rows[j % R], acc.at[dstb.at[pl.ds(goff + j * STEP, STEP)]],
                ssem[j % 2], add=True)
            nj = j + R
            if nj < KCH:
                scp[nj - R].wait()
                cp[nj] = gather(nj, rows[j % R], gsem[j % R])
        for j in range(KCH - R, KCH):
            scp[j].wait()
        return 0
    lax.fori_loop(0, nchunks, chunk, 0)

    plsc.subcore_barrier()
    r0 = s * stripe

    def cp(z, _):
        pltpu.sync_copy(acc.at[pl.ds(r0 + z * 128, 128)], rows1)
        pltpu.sync_copy(rows1, out.at[c, pl.ds(r0 + z * 128, 128)])
        return 0
    lax.fori_loop(0, stripe // 128, cp, 0)


@functools.partial(jax.jit, static_argnums=(3,))
def _lap_call(table, src_i, dst_i, acc_rows):
    stripe = acc_rows // NS
    hw = table.shape[1]
    nsteps = src_i.shape[0] // (NC * NS * STEP)
    mesh = plsc.VectorSubcoreMesh(core_axis_name="c", subcore_axis_name="s",
                                  num_cores=NC, num_subcores=NS)
    kern = pl.kernel(
        functools.partial(_lap_kernel, stripe, nsteps),
        out_type=jax.ShapeDtypeStruct((NC, acc_rows, hw), jnp.bfloat16),
        mesh=mesh,
        scratch_types=[
            pltpu.VMEM_SHARED((acc_rows, hw), jnp.bfloat16),
            pltpu.VMEM((2 * KCH * STEP,), jnp.int32),
            pltpu.VMEM((2 * KCH * STEP,), jnp.int32),
            pltpu.VMEM((128, hw), jnp.bfloat16),
            pltpu.VMEM((128, hw), jnp.bfloat16),
            pltpu.VMEM((128, hw), jnp.bfloat16),
            pltpu.VMEM((128, hw), jnp.bfloat16),
            pltpu.VMEM((128, hw), jnp.bfloat16),
            pltpu.SemaphoreType.DMA,
            pltpu.SemaphoreType.DMA,
            pltpu.SemaphoreType.DMA,
            pltpu.SemaphoreType.DMA,
            pltpu.SemaphoreType.DMA,
            pltpu.SemaphoreType.DMA,
            pltpu.SemaphoreType.DMA,
            pltpu.SemaphoreType.DMA,
            pltpu.SemaphoreType.DMA,
        ],
        compiler_params=pltpu.CompilerParams(use_tc_tiling_on_sc=False),
    )
    return kern(table, src_i, dst_i)


def _relu_bn(x):
    return jnp.maximum(x * S0, 0.0)


def _dinv_of(deg_r):
    deg = deg_r[0] + deg_r[1]
    return jnp.where(deg > 0, lax.rsqrt(jnp.maximum(deg, 1.0)), 0.0)


def _prologue(Tb, tfb, WoT, bo, wh, bh, wt, bt, WsT, bs):
    Thist = _relu_bn(jnp.dot(Tb, WoT, preferred_element_type=jnp.float32) + bo)
    Tst = _relu_bn(Tb[:, -1:] * wh + bh)
    pre = jnp.dot(tfb, wt, preferred_element_type=jnp.float32) + bt
    te = pre * jax.nn.sigmoid(pre)
    Tst = Tst + te * WsT + bs
    return Thist, Tst


def _reaction(Thist, comb, HEaT, HEbT, heb, K1T, k1b, K2T, k2b, KUT, kub):
    Th = _relu_bn(jnp.dot(Thist, HEaT, preferred_element_type=jnp.float32)
                  + jnp.dot(comb, HEbT, preferred_element_type=jnp.float32) + heb)
    dT = (jnp.dot(Th, K1T, preferred_element_type=jnp.float32) + k1b
          + jnp.dot(Thist, KUT, preferred_element_type=jnp.float32) + kub
          + comb * jnp.clip(jnp.dot(Th, K2T, preferred_element_type=jnp.float32)
                            + k2b, -1.0, 1.0))
    return _relu_bn(comb + H * dT)


def _diffusion(Rst, ay_r, dinv, Kd):
    ay = (ay_r[0] + ay_r[1]).astype(jnp.float32) * dinv
    return Rst - H * Kd * (Rst - ay)


def _tc1_body(T_r, tf_r, deg_r, WoT_r, bo_r, wh_r, bh_r, wt_r, bt_r, WsT_r,
              bs_r, C0_r, HEaT_r, HEbT_r, heb_r, K1T_r, k1b_r, K2T_r, k2b_r,
              KUT_r, kub_r, rst_out, rs2_out):
    Tb = T_r[...]
    Thist, Tst = _prologue(Tb, tf_r[...], WoT_r[...], bo_r[...], wh_r[...],
                           bh_r[...], wt_r[...], bt_r[...], WsT_r[...],
                           bs_r[...])
    comb = Tst * C0_r[:, 0:1]
    Rst = _reaction(Thist, comb, HEaT_r[...], HEbT_r[...], heb_r[...],
                    K1T_r[...], k1b_r[...], K2T_r[...], k2b_r[...],
                    KUT_r[...], kub_r[...])
    rst_out[...] = Rst
    dinv = _dinv_of(deg_r)
    rs2_out[...] = (Rst * dinv).astype(jnp.bfloat16)


def _tc2_body(T_r, tf_r, deg_r, rst0_r, ay_r, WoT_r, bo_r, wh_r, bh_r, wt_r,
              bt_r, WsT_r, bs_r, C1_r, Kd0_r, RSaT_r, RSbT_r, rsb_r, HEaT_r,
              HEbT_r, heb_r, K1T_r, k1b_r, K2T_r, k2b_r, KUT_r, kub_r,
              rst_out, rs2_out):
    Tb = T_r[...]
    dinv = _dinv_of(deg_r)
    Dst = _diffusion(rst0_r[...], ay_r, dinv, Kd0_r[...])
    Tnew1 = _relu_bn(jnp.dot(Tb, RSaT_r[...], preferred_element_type=jnp.float32)
                     + jnp.dot(Dst, RSbT_r[...], preferred_element_type=jnp.float32)
                     + rsb_r[...])
    Thist, Tst = _prologue(Tb, tf_r[...], WoT_r[...], bo_r[...], wh_r[...],
                           bh_r[...], wt_r[...], bt_r[...], WsT_r[...],
                           bs_r[...])
    comb = Tst * C1_r[:, 0:1] + Tnew1 * C1_r[:, 1:2]
    Rst = _reaction(Thist, comb, HEaT_r[...], HEbT_r[...], heb_r[...],
                    K1T_r[...], k1b_r[...], K2T_r[...], k2b_r[...],
                    KUT_r[...], kub_r[...])
    rst_out[...] = Rst
    rs2_out[...] = (Rst * dinv).astype(jnp.bfloat16)


def _tc3_body(T_r, deg_r, rst1_r, ay_r, Kd1_r, RSaT_r, RSbT_r, rsb_r, WcT_r,
              bc_r, out_r):
    Tb = T_r[...]
    dinv = _dinv_of(deg_r)
    Dst = _diffusion(rst1_r[...], ay_r, dinv, Kd1_r[...])
    Tnew2 = _relu_bn(jnp.dot(Tb, RSaT_r[...], preferred_element_type=jnp.float32)
                     + jnp.dot(Dst, RSbT_r[...], preferred_element_type=jnp.float32)
                     + rsb_r[...])
    out_r[...] = (jnp.dot(Tnew2, WcT_r[...], preferred_element_type=jnp.float32)
                  + bc_r[...])


def _wspec(shape):
    nd = len(shape)
    return pl.BlockSpec(shape, lambda i: (0,) * nd)


def _rowspec(nb, d):
    return pl.BlockSpec((nb, d), lambda i: (i, 0))


def _degspec(nb):
    return pl.BlockSpec((2, nb, 1), lambda i: (0, i, 0))


def _ayspec(nb, nhid):
    return pl.BlockSpec((2, nb, nhid), lambda i: (0, i, 0))


def kernel(T, time_feature, edge_index, W_openHist, b_openHist, w_hist, b_hist,
           w_time, b_time, W_state, b_state, KR1_W, KR1_b, KR2_W, KR2_b,
           KRU0_W, KRU0_b, Kappa, HE_W, HE_b, RS_W, RS_b, C0, C1, W_close,
           b_close):
    N = T.shape[0]
    E = edge_index.shape[1]
    nin = T.shape[1]
    nhid = W_openHist.shape[0]
    hw = nhid // 2

    NB = 1000
    while N % NB:
        NB -= 8
    nblk = N // NB

    e_pad = _round_up(E, NC * NS * STEP * KCH)
    acc_rows = _round_up(N + 1, NS * 128)
    stripe = acc_rows // NS

    src = edge_index[0]
    dst = edge_index[1]
    pad = e_pad - E
    src_g = jnp.concatenate([src, jnp.zeros((pad,), jnp.int32)])
    dst_g = jnp.concatenate([dst, jnp.full((pad,), N, jnp.int32)])
    src_d = jnp.concatenate([src, jnp.full((pad,), N, jnp.int32)])

    f32 = jnp.float32
    WoT = W_openHist.T
    bo = b_openHist[None]
    wh = w_hist[None]
    bh = b_hist[None]
    wt = w_time[:, None]
    bt = b_time[None]
    WsT = W_state.T
    bs = b_state[None]
    C0m = C0[None].astype(f32)
    C1m = C1[None].astype(f32)
    Kd = jnp.clip(Kappa, 0.0, 1.0)
    HEaT = [(HE_W[i][:, :nhid] + HE_W[i][:, 2 * nhid:]).T for i in range(2)]
    HEbT = [HE_W[i][:, nhid:2 * nhid].T for i in range(2)]
    heb = [HE_b[i][None] for i in range(2)]
    K1T = [KR1_W[i].T for i in range(2)]
    k1b = [KR1_b[i][None] for i in range(2)]
    K2T = [KR2_W[i].T for i in range(2)]
    k2b = [KR2_b[i][None] for i in range(2)]
    KUT = [KRU0_W[i].T for i in range(2)]
    kub = [KRU0_b[i][None] for i in range(2)]
    RSaT = [RS_W[i][:, :nin].T for i in range(2)]
    RSbT = [RS_W[i][:, nin:].T for i in range(2)]
    rsb = [RS_b[i][None] for i in range(2)]
    WcT = W_close.T
    bc = b_close[None]

    tf2 = time_feature.reshape(N, -1)
    nfreq = tf2.shape[1]

    deg_p = _deg_call(src_d, acc_rows)
    deg_tc = deg_p[:, :, 0:1]

    cparams = pltpu.CompilerParams(dimension_semantics=("arbitrary",))

    w1 = [WoT, bo, wh, bh, wt, bt, WsT, bs, C0m, HEaT[0], HEbT[0], heb[0],
          K1T[0], k1b[0], K2T[0], k2b[0], KUT[0], kub[0]]
    rst0, rs0 = pl.pallas_call(
        _tc1_body,
        grid=(nblk,),
        in_specs=[_rowspec(NB, nin), _rowspec(NB, nfreq), _degspec(NB)]
                 + [_wspec(w.shape) for w in w1],
        out_specs=[_rowspec(NB, nhid), _rowspec(NB, nhid)],
        out_shape=[jax.ShapeDtypeStruct((N, nhid), f32),
                   jax.ShapeDtypeStruct((N, nhid), jnp.bfloat16)],
        compiler_params=cparams,
    )(T, tf2, deg_tc, *w1)

    ay0_tc = _lap_call(rs0, src_g, dst_g, acc_rows)

    w2 = [WoT, bo, wh, bh, wt, bt, WsT, bs, C1m, Kd[0:1], RSaT[0], RSbT[0],
          rsb[0], HEaT[1], HEbT[1], heb[1], K1T[1], k1b[1], K2T[1], k2b[1],
          KUT[1], kub[1]]
    rst1, rs1 = pl.pallas_call(
        _tc2_body,
        grid=(nblk,),
        in_specs=[_rowspec(NB, nin), _rowspec(NB, nfreq), _degspec(NB),
                  _rowspec(NB, nhid), _ayspec(NB, nhid)]
                 + [_wspec(w.shape) for w in w2],
        out_specs=[_rowspec(NB, nhid), _rowspec(NB, nhid)],
        out_shape=[jax.ShapeDtypeStruct((N, nhid), f32),
                   jax.ShapeDtypeStruct((N, nhid), jnp.bfloat16)],
        compiler_params=cparams,
    )(T, tf2, deg_tc, rst0, ay0_tc, *w2)

    ay1_tc = _lap_call(rs1, src_g, dst_g, acc_rows)

    nout = W_close.shape[0]
    w3 = [Kd[1:2], RSaT[1], RSbT[1], rsb[1], WcT, bc]
    out = pl.pallas_call(
        _tc3_body,
        grid=(nblk,),
        in_specs=[_rowspec(NB, nin), _degspec(NB), _rowspec(NB, nhid),
                  _ayspec(NB, nhid)] + [_wspec(w.shape) for w in w3],
        out_specs=_rowspec(NB, nout),
        out_shape=jax.ShapeDtypeStruct((N, nout), f32),
        compiler_params=cparams,
    )(T, deg_tc, rst1, ay1_tc, *w3)
    return out


# speedup vs baseline: 16.0050x; 1.0370x over previous
import functools

import jax
import jax.numpy as jnp
from jax import lax
from jax.experimental import pallas as pl
from jax.experimental.pallas import tpu as pltpu
from jax.experimental.pallas import tpu_sc as plsc

NC = 2
NS = 16
L = 16

H = 0.1
S0 = float((1.0 + 1e-5) ** -0.5)

STEP = 128
KCH = 8


def _round_up(x, m):
    return (x + m - 1) // m * m


def _fill(buf, val):
    lanes = 32 if buf.dtype == jnp.bfloat16 else L
    v = jnp.full((lanes,), val, buf.dtype)
    ncol = buf.shape[1]

    def body(i, _):
        for k in range(ncol // lanes):
            buf[i, pl.ds(k * lanes, lanes)] = v
        return 0
    lax.fori_loop(0, buf.shape[0], body, 0)


def _deg_kernel(stripe, nsteps, src_i, deg_out, acc, idxb, onesb, sem0):
    c = lax.axis_index("c")
    s = lax.axis_index("s")
    _fill(onesb, 0.0)
    for z in range(stripe // 128):
        pltpu.sync_copy(onesb, acc.at[pl.ds(s * stripe + z * 128, 128)])
    _fill(onesb, 1.0)
    plsc.subcore_barrier()

    ebase = (c * NS + s) * nsteps * STEP

    def chunk(g, _):
        pltpu.sync_copy(src_i.at[pl.ds(ebase + g * (KCH * STEP), KCH * STEP)],
                        idxb)
        cps = [pltpu.async_copy(onesb, acc.at[idxb.at[pl.ds(j * STEP, STEP)]],
                                sem0, add=True) for j in range(KCH)]
        for cp in cps:
            cp.wait()
        return 0
    lax.fori_loop(0, nsteps // KCH, chunk, 0)

    plsc.subcore_barrier()
    r0 = s * stripe

    def cp(z, _):
        pltpu.sync_copy(acc.at[pl.ds(r0 + z * 128, 128)], onesb)
        pltpu.sync_copy(onesb, deg_out.at[c, pl.ds(r0 + z * 128, 128)])
        return 0
    lax.fori_loop(0, stripe // 128, cp, 0)


@functools.partial(jax.jit, static_argnums=(1,))
def _deg_call(src_i, nrows):
    stripe = nrows // NS
    nsteps = src_i.shape[0] // (NC * NS * STEP)
    mesh = plsc.VectorSubcoreMesh(core_axis_name="c", subcore_axis_name="s",
                                  num_cores=NC, num_subcores=NS)
    kern = pl.kernel(
        functools.partial(_deg_kernel, stripe, nsteps),
        out_type=jax.ShapeDtypeStruct((NC, nrows, 32), jnp.float32),
        mesh=mesh,
        scratch_types=[
            pltpu.VMEM_SHARED((nrows, 32), jnp.float32),
            pltpu.VMEM((KCH * STEP,), jnp.int32),
            pltpu.VMEM((128, 32), jnp.float32),
            pltpu.SemaphoreType.DMA,
        ],
        compiler_params=pltpu.CompilerParams(use_tc_tiling_on_sc=False),
    )
    return kern(src_i)


def _lap_kernel(stripe, k0, k1, table, src_i, dst_i, out, acc, srcb, dstb,
                rows0, rows1, rows2, rows3, rows4, sem0, sem1, sem2, sem3,
                sem4, ssem0, ssem1, semis, semid):
    c = lax.axis_index("c")
    s = lax.axis_index("s")

    _fill(rows0, 0.0)
    for z in range(stripe // 128):
        pltpu.sync_copy(rows0, acc.at[pl.ds(s * stripe + z * 128, 128)])
    plsc.subcore_barrier()

    CL = KCH * STEP
    kc = jnp.where(c == 0, k0, k1)
    nchunks = kc
    ebase = (c * (NS * k0) + s * kc) * CL
    rows = (rows0, rows1, rows2, rows3, rows4)
    gsem = (sem0, sem1, sem2, sem3, sem4)
    ssem = (ssem0, ssem1)
    R = len(rows)

    pltpu.async_copy(src_i.at[pl.ds(ebase, CL)], srcb.at[pl.ds(0, CL)], semis)
    pltpu.async_copy(dst_i.at[pl.ds(ebase, CL)], dstb.at[pl.ds(0, CL)], semid)

    def chunk(g, _):
        coff = ebase + g * CL
        goff = lax.rem(g, 2) * CL
        noff = CL - goff
        pltpu.make_async_copy(src_i.at[pl.ds(coff, CL)],
                              srcb.at[pl.ds(goff, CL)], semis).wait()
        pltpu.make_async_copy(dst_i.at[pl.ds(coff, CL)],
                              dstb.at[pl.ds(goff, CL)], semid).wait()

        @pl.when(g + 1 < nchunks)
        def _():
            pltpu.async_copy(src_i.at[pl.ds(coff + CL, CL)],
                             srcb.at[pl.ds(noff, CL)], semis)
            pltpu.async_copy(dst_i.at[pl.ds(coff + CL, CL)],
                             dstb.at[pl.ds(noff, CL)], semid)

        def gather(j, buf, sem):
            return pltpu.async_copy(
                table.at[srcb.at[pl.ds(goff + j * STEP, STEP)]], buf, sem)

        cp = [None] * KCH
        scp = [None] * KCH
        for j in range(R):
            cp[j] = gather(j, rows[j], gsem[j])
        for j in range(KCH):
            cp[j].wait()
            scp[j] = pltpu.async_copy(
                rows[j % R], acc.at[dstb.at[pl.ds(goff + j * STEP, STEP)]],
                ssem[j % 2], add=True)
            nj = j + R
            if nj < KCH:
                scp[nj - R].wait()
                cp[nj] = gather(nj, rows[j % R], gsem[j % R])
        for j in range(KCH - R, KCH):
            scp[j].wait()
        return 0
    lax.fori_loop(0, nchunks, chunk, 0)

    plsc.subcore_barrier()
    r0 = s * stripe

    def cp(z, _):
        pltpu.sync_copy(acc.at[pl.ds(r0 + z * 128, 128)], rows1)
        pltpu.sync_copy(rows1, out.at[c, pl.ds(r0 + z * 128, 128)])
        return 0
    lax.fori_loop(0, stripe // 128, cp, 0)


SPLIT0 = 0.72


@functools.partial(jax.jit, static_argnums=(3,))
def _lap_call(table, src_i, dst_i, acc_rows):
    stripe = acc_rows // NS
    hw = table.shape[1]
    cpp = src_i.shape[0] // (NS * KCH * STEP)
    k0 = min(cpp - 1, max(1, int(round(cpp * SPLIT0))))
    k1 = cpp - k0
    mesh = plsc.VectorSubcoreMesh(core_axis_name="c", subcore_axis_name="s",
                                  num_cores=NC, num_subcores=NS)
    kern = pl.kernel(
        functools.partial(_lap_kernel, stripe, k0, k1),
        out_type=jax.ShapeDtypeStruct((NC, acc_rows, hw), jnp.bfloat16),
        mesh=mesh,
        scratch_types=[
            pltpu.VMEM_SHARED((acc_rows, hw), jnp.bfloat16),
            pltpu.VMEM((2 * KCH * STEP,), jnp.int32),
            pltpu.VMEM((2 * KCH * STEP,), jnp.int32),
            pltpu.VMEM((128, hw), jnp.bfloat16),
            pltpu.VMEM((128, hw), jnp.bfloat16),
            pltpu.VMEM((128, hw), jnp.bfloat16),
            pltpu.VMEM((128, hw), jnp.bfloat16),
            pltpu.VMEM((128, hw), jnp.bfloat16),
            pltpu.SemaphoreType.DMA,
            pltpu.SemaphoreType.DMA,
            pltpu.SemaphoreType.DMA,
            pltpu.SemaphoreType.DMA,
            pltpu.SemaphoreType.DMA,
            pltpu.SemaphoreType.DMA,
            pltpu.SemaphoreType.DMA,
            pltpu.SemaphoreType.DMA,
            pltpu.SemaphoreType.DMA,
        ],
        compiler_params=pltpu.CompilerParams(use_tc_tiling_on_sc=False),
    )
    return kern(table, src_i, dst_i)


def _relu_bn(x):
    return jnp.maximum(x * S0, 0.0)


def _dinv_of(deg_r):
    deg = deg_r[0] + deg_r[1]
    return jnp.where(deg > 0, lax.rsqrt(jnp.maximum(deg, 1.0)), 0.0)


def _prologue(Tb, tfb, WoT, bo, wh, bh, wt, bt, WsT, bs):
    Thist = _relu_bn(jnp.dot(Tb, WoT, preferred_element_type=jnp.float32) + bo)
    Tst = _relu_bn(Tb[:, -1:] * wh + bh)
    pre = jnp.dot(tfb, wt, preferred_element_type=jnp.float32) + bt
    te = pre * jax.nn.sigmoid(pre)
    Tst = Tst + te * WsT + bs
    return Thist, Tst


def _reaction(Thist, comb, HEaT, HEbT, heb, K1T, k1b, K2T, k2b, KUT, kub):
    Th = _relu_bn(jnp.dot(Thist, HEaT, preferred_element_type=jnp.float32)
                  + jnp.dot(comb, HEbT, preferred_element_type=jnp.float32) + heb)
    dT = (jnp.dot(Th, K1T, preferred_element_type=jnp.float32) + k1b
          + jnp.dot(Thist, KUT, preferred_element_type=jnp.float32) + kub
          + comb * jnp.clip(jnp.dot(Th, K2T, preferred_element_type=jnp.float32)
                            + k2b, -1.0, 1.0))
    return _relu_bn(comb + H * dT)


def _diffusion(Rst, ay_r, dinv, Kd):
    ay = (ay_r[0] + ay_r[1]).astype(jnp.float32) * dinv
    return Rst - H * Kd * (Rst - ay)


def _tc1_body(T_r, tf_r, deg_r, WoT_r, bo_r, wh_r, bh_r, wt_r, bt_r, WsT_r,
              bs_r, C0_r, HEaT_r, HEbT_r, heb_r, K1T_r, k1b_r, K2T_r, k2b_r,
              KUT_r, kub_r, rst_out, rs2_out):
    Tb = T_r[...]
    Thist, Tst = _prologue(Tb, tf_r[...], WoT_r[...], bo_r[...], wh_r[...],
                           bh_r[...], wt_r[...], bt_r[...], WsT_r[...],
                           bs_r[...])
    comb = Tst * C0_r[:, 0:1]
    Rst = _reaction(Thist, comb, HEaT_r[...], HEbT_r[...], heb_r[...],
                    K1T_r[...], k1b_r[...], K2T_r[...], k2b_r[...],
                    KUT_r[...], kub_r[...])
    rst_out[...] = Rst
    dinv = _dinv_of(deg_r)
    rs2_out[...] = (Rst * dinv).astype(jnp.bfloat16)


def _tc2_body(T_r, tf_r, deg_r, rst0_r, ay_r, WoT_r, bo_r, wh_r, bh_r, wt_r,
              bt_r, WsT_r, bs_r, C1_r, Kd0_r, RSaT_r, RSbT_r, rsb_r, HEaT_r,
              HEbT_r, heb_r, K1T_r, k1b_r, K2T_r, k2b_r, KUT_r, kub_r,
              rst_out, rs2_out):
    Tb = T_r[...]
    dinv = _dinv_of(deg_r)
    Dst = _diffusion(rst0_r[...], ay_r, dinv, Kd0_r[...])
    Tnew1 = _relu_bn(jnp.dot(Tb, RSaT_r[...], preferred_element_type=jnp.float32)
                     + jnp.dot(Dst, RSbT_r[...], preferred_element_type=jnp.float32)
                     + rsb_r[...])
    Thist, Tst = _prologue(Tb, tf_r[...], WoT_r[...], bo_r[...], wh_r[...],
                           bh_r[...], wt_r[...], bt_r[...], WsT_r[...],
                           bs_r[...])
    comb = Tst * C1_r[:, 0:1] + Tnew1 * C1_r[:, 1:2]
    Rst = _reaction(Thist, comb, HEaT_r[...], HEbT_r[...], heb_r[...],
                    K1T_r[...], k1b_r[...], K2T_r[...], k2b_r[...],
                    KUT_r[...], kub_r[...])
    rst_out[...] = Rst
    rs2_out[...] = (Rst * dinv).astype(jnp.bfloat16)


def _tc3_body(T_r, deg_r, rst1_r, ay_r, Kd1_r, RSaT_r, RSbT_r, rsb_r, WcT_r,
              bc_r, out_r):
    Tb = T_r[...]
    dinv = _dinv_of(deg_r)
    Dst = _diffusion(rst1_r[...], ay_r, dinv, Kd1_r[...])
    Tnew2 = _relu_bn(jnp.dot(Tb, RSaT_r[...], preferred_element_type=jnp.float32)
                     + jnp.dot(Dst, RSbT_r[...], preferred_element_type=jnp.float32)
                     + rsb_r[...])
    out_r[...] = (jnp.dot(Tnew2, WcT_r[...], preferred_element_type=jnp.float32)
                  + bc_r[...])


def _wspec(shape):
    nd = len(shape)
    return pl.BlockSpec(shape, lambda i: (0,) * nd)


def _rowspec(nb, d):
    return pl.BlockSpec((nb, d), lambda i: (i, 0))


def _degspec(nb):
    return pl.BlockSpec((2, nb, 1), lambda i: (0, i, 0))


def _ayspec(nb, nhid):
    return pl.BlockSpec((2, nb, nhid), lambda i: (0, i, 0))


def kernel(T, time_feature, edge_index, W_openHist, b_openHist, w_hist, b_hist,
           w_time, b_time, W_state, b_state, KR1_W, KR1_b, KR2_W, KR2_b,
           KRU0_W, KRU0_b, Kappa, HE_W, HE_b, RS_W, RS_b, C0, C1, W_close,
           b_close):
    N = T.shape[0]
    E = edge_index.shape[1]
    nin = T.shape[1]
    nhid = W_openHist.shape[0]
    hw = nhid // 2

    NB = 1000
    while N % NB:
        NB -= 8
    nblk = N // NB

    e_pad = _round_up(E, NC * NS * STEP * KCH)
    acc_rows = _round_up(N + 1, NS * 128)
    stripe = acc_rows // NS

    src = edge_index[0]
    dst = edge_index[1]
    pad = e_pad - E
    src_g = jnp.concatenate([src, jnp.zeros((pad,), jnp.int32)])
    dst_g = jnp.concatenate([dst, jnp.full((pad,), N, jnp.int32)])
    src_d = jnp.concatenate([src, jnp.full((pad,), N, jnp.int32)])

    f32 = jnp.float32
    WoT = W_openHist.T
    bo = b_openHist[None]
    wh = w_hist[None]
    bh = b_hist[None]
    wt = w_time[:, None]
    bt = b_time[None]
    WsT = W_state.T
    bs = b_state[None]
    C0m = C0[None].astype(f32)
    C1m = C1[None].astype(f32)
    Kd = jnp.clip(Kappa, 0.0, 1.0)
    HEaT = [(HE_W[i][:, :nhid] + HE_W[i][:, 2 * nhid:]).T for i in range(2)]
    HEbT = [HE_W[i][:, nhid:2 * nhid].T for i in range(2)]
    heb = [HE_b[i][None] for i in range(2)]
    K1T = [KR1_W[i].T for i in range(2)]
    k1b = [KR1_b[i][None] for i in range(2)]
    K2T = [KR2_W[i].T for i in range(2)]
    k2b = [KR2_b[i][None] for i in range(2)]
    KUT = [KRU0_W[i].T for i in range(2)]
    kub = [KRU0_b[i][None] for i in range(2)]
    RSaT = [RS_W[i][:, :nin].T for i in range(2)]
    RSbT = [RS_W[i][:, nin:].T for i in range(2)]
    rsb = [RS_b[i][None] for i in range(2)]
    WcT = W_close.T
    bc = b_close[None]

    tf2 = time_feature.reshape(N, -1)
    nfreq = tf2.shape[1]

    deg_p = _deg_call(src_d, acc_rows)
    deg_tc = deg_p[:, :, 0:1]

    cparams = pltpu.CompilerParams(dimension_semantics=("arbitrary",))

    w1 = [WoT, bo, wh, bh, wt, bt, WsT, bs, C0m, HEaT[0], HEbT[0], heb[0],
          K1T[0], k1b[0], K2T[0], k2b[0], KUT[0], kub[0]]
    rst0, rs0 = pl.pallas_call(
        _tc1_body,
        grid=(nblk,),
        in_specs=[_rowspec(NB, nin), _rowspec(NB, nfreq), _degspec(NB)]
                 + [_wspec(w.shape) for w in w1],
        out_specs=[_rowspec(NB, nhid), _rowspec(NB, nhid)],
        out_shape=[jax.ShapeDtypeStruct((N, nhid), f32),
                   jax.ShapeDtypeStruct((N, nhid), jnp.bfloat16)],
        compiler_params=cparams,
    )(T, tf2, deg_tc, *w1)

    ay0_tc = _lap_call(rs0, src_g, dst_g, acc_rows)

    w2 = [WoT, bo, wh, bh, wt, bt, WsT, bs, C1m, Kd[0:1], RSaT[0], RSbT[0],
          rsb[0], HEaT[1], HEbT[1], heb[1], K1T[1], k1b[1], K2T[1], k2b[1],
          KUT[1], kub[1]]
    rst1, rs1 = pl.pallas_call(
        _tc2_body,
        grid=(nblk,),
        in_specs=[_rowspec(NB, nin), _rowspec(NB, nfreq), _degspec(NB),
                  _rowspec(NB, nhid), _ayspec(NB, nhid)]
                 + [_wspec(w.shape) for w in w2],
        out_specs=[_rowspec(NB, nhid), _rowspec(NB, nhid)],
        out_shape=[jax.ShapeDtypeStruct((N, nhid), f32),
                   jax.ShapeDtypeStruct((N, nhid), jnp.bfloat16)],
        compiler_params=cparams,
    )(T, tf2, deg_tc, rst0, ay0_tc, *w2)

    ay1_tc = _lap_call(rs1, src_g, dst_g, acc_rows)

    nout = W_close.shape[0]
    w3 = [Kd[1:2], RSaT[1], RSbT[1], rsb[1], WcT, bc]
    out = pl.pallas_call(
        _tc3_body,
        grid=(nblk,),
        in_specs=[_rowspec(NB, nin), _degspec(NB), _rowspec(NB, nhid),
                  _ayspec(NB, nhid)] + [_wspec(w.shape) for w in w3],
        out_specs=_rowspec(NB, nout),
        out_shape=jax.ShapeDtypeStruct((N, nout), f32),
        compiler_params=cparams,
    )(T, deg_tc, rst1, ay1_tc, *w3)
    return out
